# baseline (speedup 1.0000x reference)
import jax
import jax.numpy as jnp
from jax.experimental import pallas as pl
from jax.experimental.pallas import tpu as pltpu

_COST_CLASS = 2.0
_COST_BBOX = 5.0
_COST_GIOU = 2.0
_ALPHA = 0.25
_GAMMA = 2.0
_OTA_K = 5
_TQ = 512


def _gt_terms(gtT, img):
    gx0, gy0, gx1, gy1 = gtT[0:1, :], gtT[1:2, :], gtT[2:3, :], gtT[3:4, :]
    gcx = (gx0 + gx1) * 0.5
    gcy = (gy0 + gy1) * 0.5
    gw = gx1 - gx0
    gh = gy1 - gy0
    xx0 = gcx - 0.5 * gw
    yy0 = gcy - 0.5 * gh
    xx1 = gcx + 0.5 * gw
    yy1 = gcy + 0.5 * gh
    area_b = (gx1 - gx0) * (gy1 - gy0)
    i0 = img[0:1, 0:1]
    i1 = img[0:1, 1:2]
    i2 = img[0:1, 2:3]
    i3 = img[0:1, 3:4]
    tb = (gx0 / i0, gy0 / i1, gx1 / i2, gy1 / i3)
    return (gx0, gy0, gx1, gy1, gcx, gcy, xx0, yy0, xx1, yy1, area_b, tb,
            i0, i1, i2, i3)


def _frame_tile(bx, score, lgT, oh, g):
    (gx0, gy0, gx1, gy1, gcx, gcy, xx0, yy0, xx1, yy1, area_b, tb,
     i0, i1, i2, i3) = g
    x0, y0, x1, y1 = bx[:, 0:1], bx[:, 1:2], bx[:, 2:3], bx[:, 3:4]
    cx = (x0 + x1) * 0.5
    cy = (y0 + y1) * 0.5

    in_boxes = (cx > xx0) & (cx < xx1) & (cy > yy0) & (cy < yy1)
    in_boxes_all = jnp.sum(in_boxes.astype(jnp.int32), axis=1, keepdims=True) > 0
    r = 2.5
    w_rt = xx1 - xx0
    h_rt = yy1 - yy0
    in_centers = ((cx > gcx - r * w_rt) & (cx < gcx + r * w_rt)
                  & (cy > gcy - r * h_rt) & (cy < gcy + r * h_rt))
    in_centers_all = jnp.sum(in_centers.astype(jnp.int32), axis=1, keepdims=True) > 0
    fg = in_boxes_all | in_centers_all
    both = in_boxes & in_centers

    area_a = (x1 - x0) * (y1 - y0)
    lx = jnp.maximum(x0, gx0)
    ly = jnp.maximum(y0, gy0)
    rx = jnp.minimum(x1, gx1)
    ry = jnp.minimum(y1, gy1)
    iw = jnp.maximum(rx - lx, 0.0)
    ih = jnp.maximum(ry - ly, 0.0)
    inter = iw * ih
    union = area_a + area_b - inter
    iou = inter / (union + 1e-8)
    l2x = jnp.minimum(x0, gx0)
    l2y = jnp.minimum(y0, gy0)
    r2x = jnp.maximum(x1, gx1)
    r2y = jnp.maximum(y1, gy1)
    w2 = jnp.maximum(r2x - l2x, 0.0)
    h2 = jnp.maximum(r2y - l2y, 0.0)
    area_c = w2 * h2
    giou = iou - (area_c - union) / (area_c + 1e-8)

    cb = (jnp.abs(x0 / i0 - tb[0]) + jnp.abs(y0 / i1 - tb[1])
          + jnp.abs(x1 / i2 - tb[2]) + jnp.abs(y1 / i3 - tb[3]))

    lg = jax.lax.dot_general(
        lgT, oh, (((0,), (0,)), ((), ())),
        precision=jax.lax.Precision.HIGHEST,
        preferred_element_type=jnp.float32)
    p = jnp.sqrt(jax.nn.sigmoid(lg) * score)
    neg = (1.0 - _ALPHA) * (p * p) * -jnp.log(1.0 - p + 1e-8)
    pos = _ALPHA * ((1.0 - p) * (1.0 - p)) * -jnp.log(p + 1e-8)
    d = pos - neg
    return fg, both, iou, giou, cb, d


def _body(lp_ref, lc_ref, pq_ref, idp_ref, idc_ref,
          gp_ref, gc_ref, imp_ref, imc_ref,
          matching_ref, selgt_ref, mqidx_ref, cost_s, tmp_s):
    Q, C = lp_ref.shape[2], lp_ref.shape[1]
    G = idp_ref.shape[2]
    NT = Q // _TQ

    gterms_p = _gt_terms(gp_ref[0], imp_ref[0])
    gterms_c = _gt_terms(gc_ref[0], imc_ref[0])
    oh_p = (jax.lax.broadcasted_iota(jnp.int32, (C, G), 0)
            == idp_ref[0]).astype(jnp.float32)
    oh_c = (jax.lax.broadcasted_iota(jnp.int32, (C, G), 0)
            == idc_ref[0]).astype(jnp.float32)
    iota_tq = jax.lax.broadcasted_iota(jnp.int32, (_TQ, 1), 0)
    iota_g = jax.lax.broadcasted_iota(jnp.int32, (1, G), 1)

    def build(t, _):
        sl = pl.ds(t * _TQ, _TQ)
        bx_p = pq_ref[0, sl, 0:4]
        bx_c = pq_ref[0, sl, 4:8]
        score = pq_ref[0, sl, 8:9]
        fg_p, both_p, iou_p, giou_p, cb_p, d_p = _frame_tile(
            bx_p, score, lp_ref[0, :, sl], oh_p, gterms_p)
        fg_c, both_c, iou_c, giou_c, cb_c, d_c = _frame_tile(
            bx_c, score, lc_ref[0, :, sl], oh_c, gterms_c)
        fg = fg_p & fg_c
        both = both_p & both_c
        ious = (iou_p + iou_c) * 0.5
        cc = d_p + d_c
        cg = -0.5 * (giou_p + giou_c)
        cost = (_COST_BBOX * (cb_p + cb_c) * 0.5 + _COST_CLASS * cc * 0.5
                + _COST_GIOU * cg + 100.0 * jnp.where(both, 0.0, 1.0))
        cost = cost + 10000.0 * jnp.where(fg, 0.0, 1.0)
        cost_s[sl, :] = cost
        tmp_s[sl, :] = ious
        matching_ref[0, sl, :] = jnp.zeros((_TQ, G), jnp.float32)
        return 0

    jax.lax.fori_loop(0, NT, build, 0)

    def col_extreme(ref_read, is_min):
        init_v = jnp.full((1, G), jnp.inf if is_min else -jnp.inf, jnp.float32)
        init_i = jnp.full((1, G), Q, jnp.int32)

        def scan(t, carry):
            v, i = carry
            sl = pl.ds(t * _TQ, _TQ)
            tile = ref_read(sl)
            if is_min:
                tv = jnp.min(tile, axis=0, keepdims=True)
                better = tv < v
            else:
                tv = jnp.max(tile, axis=0, keepdims=True)
                better = tv > v
            gi = iota_tq + t * _TQ
            ti = jnp.min(jnp.where(tile == tv, gi, Q), axis=0, keepdims=True)
            nv = jnp.where(better, tv, v)
            ni = jnp.where(better, ti, jnp.where(tv == v, jnp.minimum(i, ti), i))
            return nv, ni

        return jax.lax.fori_loop(0, NT, scan, (init_v, init_i))

    s = jnp.zeros((1, G), jnp.float32)
    for _ in range(_OTA_K):
        m, idx = col_extreme(lambda sl: tmp_s[sl, :], is_min=False)
        s = s + m

        def mask_out(t, _):
            sl = pl.ds(t * _TQ, _TQ)
            gi = iota_tq + t * _TQ
            tmp_s[sl, :] = jnp.where(gi == idx, -jnp.inf, tmp_s[sl, :])
            return 0

        jax.lax.fori_loop(0, NT, mask_out, 0)
    dynamic_ks = jnp.maximum(s.astype(jnp.int32), 1)

    def copy_cost(t, _):
        sl = pl.ds(t * _TQ, _TQ)
        tmp_s[sl, :] = cost_s[sl, :]
        return 0

    jax.lax.fori_loop(0, NT, copy_cost, 0)

    for j in range(_OTA_K):
        m, idx = col_extreme(lambda sl: tmp_s[sl, :], is_min=True)
        jhit = j < dynamic_ks

        def set_match(t, _, idx=idx, jhit=jhit):
            sl = pl.ds(t * _TQ, _TQ)
            gi = iota_tq + t * _TQ
            hit = (gi == idx) & jhit
            matching_ref[0, sl, :] = jnp.where(hit, 1.0, matching_ref[0, sl, :])
            tmp_s[sl, :] = jnp.where(gi == idx, jnp.inf, tmp_s[sl, :])
            return 0

        jax.lax.fori_loop(0, NT, set_match, 0)

    def row_resolve_terms(sl):
        cost = cost_s[sl, :]
        rmin = jnp.min(cost, axis=1, keepdims=True)
        row_argmin = jnp.min(jnp.where(cost == rmin, iota_g, G),
                             axis=1, keepdims=True)
        oh_row = (iota_g == row_argmin).astype(jnp.float32)
        return cost, oh_row

    def pass_a(t, colsum):
        sl = pl.ds(t * _TQ, _TQ)
        m_tile = matching_ref[0, sl, :]
        _, oh_row = row_resolve_terms(sl)
        amg = jnp.sum(m_tile, axis=1, keepdims=True)
        newm = jnp.where(amg > 1, oh_row, m_tile)
        matching_ref[0, sl, :] = newm
        return colsum + jnp.sum(newm, axis=0, keepdims=True)

    colsum = jax.lax.fori_loop(0, NT, pass_a, jnp.zeros((1, G), jnp.float32))
    unmatched = colsum == 0

    def pen_read(sl):
        matched_q = jnp.sum(matching_ref[0, sl, :], axis=1, keepdims=True) > 0
        return cost_s[sl, :] + matched_q.astype(jnp.float32) * 100000.0

    _, col_argmin = col_extreme(pen_read, is_min=True)

    def pass_c(t, carry):
        cmv, cmi = carry
        sl = pl.ds(t * _TQ, _TQ)
        gi = iota_tq + t * _TQ
        m_tile = matching_ref[0, sl, :]
        fix = (gi == col_argmin).astype(jnp.float32)
        m2 = jnp.where(unmatched, fix, m_tile)
        amg2 = jnp.sum(m2, axis=1, keepdims=True)
        cost, oh_row = row_resolve_terms(sl)
        m3 = jnp.where(amg2 > 1, oh_row, m2)
        matching_ref[0, sl, :] = m3
        sel = (jnp.sum(m3, axis=1, keepdims=True) > 0).astype(jnp.int32)
        rowmax = jnp.max(m3, axis=1, keepdims=True)
        mgt = jnp.min(jnp.where(m3 == rowmax, iota_g, G), axis=1, keepdims=True)
        selgt_ref[0, sl, 0:1] = sel
        selgt_ref[0, sl, 1:2] = mgt
        cmask = jnp.where(m3 > 0, cost, 1e18)
        tv = jnp.min(cmask, axis=0, keepdims=True)
        ti = jnp.min(jnp.where(cmask == tv, gi, Q), axis=0, keepdims=True)
        better = tv < cmv
        nv = jnp.where(better, tv, cmv)
        ni = jnp.where(better, ti, jnp.where(tv == cmv, jnp.minimum(cmi, ti), cmi))
        return nv, ni

    _, mqidx = jax.lax.fori_loop(
        0, NT, pass_c,
        (jnp.full((1, G), jnp.inf, jnp.float32), jnp.full((1, G), Q, jnp.int32)))
    mqidx_ref[0] = mqidx


def kernel(pred_logits, pred_boxes, pred_scores, tgt_labels, tgt_boxes_xyxy,
           image_size_xyxy):
    B, Q, C = pred_logits.shape
    B2 = B // 2
    G = tgt_labels.shape[1]

    lp = jnp.swapaxes(pred_logits[:B2], 1, 2)
    lc = jnp.swapaxes(pred_logits[B2:], 1, 2)
    pq = jnp.concatenate(
        [pred_boxes[:B2], pred_boxes[B2:], pred_scores], axis=-1)
    idp = tgt_labels[:B2].reshape(B2, 1, G)
    idc = tgt_labels[B2:].reshape(B2, 1, G)
    gp = jnp.swapaxes(tgt_boxes_xyxy[:B2], 1, 2)
    gc = jnp.swapaxes(tgt_boxes_xyxy[B2:], 1, 2)
    imp = image_size_xyxy[:B2].reshape(B2, 1, 4)
    imc = image_size_xyxy[B2:].reshape(B2, 1, 4)

    def spec(shape):
        n = len(shape)
        return pl.BlockSpec((1,) + shape[1:], lambda b: (b,) + (0,) * (n - 1))

    out_shapes = (
        jax.ShapeDtypeStruct((B2, Q, G), jnp.float32),
        jax.ShapeDtypeStruct((B2, Q, 2), jnp.int32),
        jax.ShapeDtypeStruct((B2, 1, G), jnp.int32),
    )
    args = (lp, lc, pq, idp, idc, gp, gc, imp, imc)
    matching, selgt, mqidx = pl.pallas_call(
        _body,
        grid=(B2,),
        in_specs=[spec(a.shape) for a in args],
        out_specs=tuple(spec(s.shape) for s in out_shapes),
        out_shape=out_shapes,
        scratch_shapes=[pltpu.VMEM((Q, G), jnp.float32),
                        pltpu.VMEM((Q, G), jnp.float32)],
        compiler_params=pltpu.CompilerParams(
            dimension_semantics=("parallel",)),
    )(*args)

    return (matching,
            selgt[:, :, 0].astype(bool),
            selgt[:, :, 1],
            mqidx.reshape(B2, G))


# speedup vs baseline: 1.1614x; 1.1614x over previous
import jax
import jax.numpy as jnp
from jax.experimental import pallas as pl
from jax.experimental.pallas import tpu as pltpu

_COST_CLASS = 2.0
_COST_BBOX = 5.0
_COST_GIOU = 2.0
_ALPHA = 0.25
_GAMMA = 2.0
_OTA_K = 5
_TQ = 1024


def _gt_terms(gtT, img):
    gx0, gy0, gx1, gy1 = gtT[0:1, :], gtT[1:2, :], gtT[2:3, :], gtT[3:4, :]
    gcx = (gx0 + gx1) * 0.5
    gcy = (gy0 + gy1) * 0.5
    gw = gx1 - gx0
    gh = gy1 - gy0
    xx0 = gcx - 0.5 * gw
    yy0 = gcy - 0.5 * gh
    xx1 = gcx + 0.5 * gw
    yy1 = gcy + 0.5 * gh
    area_b = (gx1 - gx0) * (gy1 - gy0)
    i0 = img[0:1, 0:1]
    i1 = img[0:1, 1:2]
    i2 = img[0:1, 2:3]
    i3 = img[0:1, 3:4]
    tb = (gx0 / i0, gy0 / i1, gx1 / i2, gy1 / i3)
    return (gx0, gy0, gx1, gy1, gcx, gcy, xx0, yy0, xx1, yy1, area_b, tb,
            i0, i1, i2, i3)


def _frame_tile(bx, score, lgT, oh, g):
    (gx0, gy0, gx1, gy1, gcx, gcy, xx0, yy0, xx1, yy1, area_b, tb,
     i0, i1, i2, i3) = g
    x0, y0, x1, y1 = bx[:, 0:1], bx[:, 1:2], bx[:, 2:3], bx[:, 3:4]
    cx = (x0 + x1) * 0.5
    cy = (y0 + y1) * 0.5

    in_boxes = (cx > xx0) & (cx < xx1) & (cy > yy0) & (cy < yy1)
    in_boxes_all = jnp.sum(in_boxes.astype(jnp.int32), axis=1, keepdims=True) > 0
    r = 2.5
    w_rt = xx1 - xx0
    h_rt = yy1 - yy0
    in_centers = ((cx > gcx - r * w_rt) & (cx < gcx + r * w_rt)
                  & (cy > gcy - r * h_rt) & (cy < gcy + r * h_rt))
    in_centers_all = jnp.sum(in_centers.astype(jnp.int32), axis=1, keepdims=True) > 0
    fg = in_boxes_all | in_centers_all
    both = in_boxes & in_centers

    area_a = (x1 - x0) * (y1 - y0)
    lx = jnp.maximum(x0, gx0)
    ly = jnp.maximum(y0, gy0)
    rx = jnp.minimum(x1, gx1)
    ry = jnp.minimum(y1, gy1)
    iw = jnp.maximum(rx - lx, 0.0)
    ih = jnp.maximum(ry - ly, 0.0)
    inter = iw * ih
    union = area_a + area_b - inter
    iou = inter / (union + 1e-8)
    l2x = jnp.minimum(x0, gx0)
    l2y = jnp.minimum(y0, gy0)
    r2x = jnp.maximum(x1, gx1)
    r2y = jnp.maximum(y1, gy1)
    w2 = jnp.maximum(r2x - l2x, 0.0)
    h2 = jnp.maximum(r2y - l2y, 0.0)
    area_c = w2 * h2
    giou = iou - (area_c - union) / (area_c + 1e-8)

    cb = (jnp.abs(x0 / i0 - tb[0]) + jnp.abs(y0 / i1 - tb[1])
          + jnp.abs(x1 / i2 - tb[2]) + jnp.abs(y1 / i3 - tb[3]))

    lg = jax.lax.dot_general(
        lgT, oh, (((0,), (0,)), ((), ())),
        precision=jax.lax.Precision.HIGHEST,
        preferred_element_type=jnp.float32)
    p = jnp.sqrt(jax.nn.sigmoid(lg) * score)
    neg = (1.0 - _ALPHA) * (p * p) * -jnp.log(1.0 - p + 1e-8)
    pos = _ALPHA * ((1.0 - p) * (1.0 - p)) * -jnp.log(p + 1e-8)
    d = pos - neg
    return fg, both, iou, giou, cb, d


def _body(lp_ref, lc_ref, pq_ref, idp_ref, idc_ref,
          gp_ref, gc_ref, imp_ref, imc_ref,
          matching_ref, selgt_ref, mqidx_ref, cost_s, cwork, iwork):
    Q, C = lp_ref.shape[2], lp_ref.shape[1]
    G = idp_ref.shape[2]
    NT = Q // _TQ

    gterms_p = _gt_terms(gp_ref[0], imp_ref[0])
    gterms_c = _gt_terms(gc_ref[0], imc_ref[0])
    oh_p = (jax.lax.broadcasted_iota(jnp.int32, (C, G), 0)
            == idp_ref[0]).astype(jnp.float32)
    oh_c = (jax.lax.broadcasted_iota(jnp.int32, (C, G), 0)
            == idc_ref[0]).astype(jnp.float32)
    iota_tq = jax.lax.broadcasted_iota(jnp.int32, (_TQ, 1), 0)
    iota_g = jax.lax.broadcasted_iota(jnp.int32, (1, G), 1)

    def fold_min(tile, gi, v, i):
        tv = jnp.min(tile, axis=0, keepdims=True)
        ti = jnp.min(jnp.where(tile == tv, gi, Q), axis=0, keepdims=True)
        better = tv < v
        nv = jnp.where(better, tv, v)
        ni = jnp.where(better, ti, jnp.where(tv == v, jnp.minimum(i, ti), i))
        return nv, ni

    def fold_max(tile, gi, v, i):
        tv = jnp.max(tile, axis=0, keepdims=True)
        ti = jnp.min(jnp.where(tile == tv, gi, Q), axis=0, keepdims=True)
        better = tv > v
        nv = jnp.where(better, tv, v)
        ni = jnp.where(better, ti, jnp.where(tv == v, jnp.minimum(i, ti), i))
        return nv, ni

    neg_inf_v = jnp.full((1, G), -jnp.inf, jnp.float32)
    pos_inf_v = jnp.full((1, G), jnp.inf, jnp.float32)
    q_idx = jnp.full((1, G), Q, jnp.int32)

    def build(t, carry):
        iv, ii, cv, ci = carry
        sl = pl.ds(t * _TQ, _TQ)
        gi = iota_tq + t * _TQ
        bx_p = pq_ref[0, sl, 0:4]
        bx_c = pq_ref[0, sl, 4:8]
        score = pq_ref[0, sl, 8:9]
        fg_p, both_p, iou_p, giou_p, cb_p, d_p = _frame_tile(
            bx_p, score, lp_ref[0, :, sl], oh_p, gterms_p)
        fg_c, both_c, iou_c, giou_c, cb_c, d_c = _frame_tile(
            bx_c, score, lc_ref[0, :, sl], oh_c, gterms_c)
        fg = fg_p & fg_c
        both = both_p & both_c
        ious = (iou_p + iou_c) * 0.5
        cc = d_p + d_c
        cg = -0.5 * (giou_p + giou_c)
        cost = (_COST_BBOX * (cb_p + cb_c) * 0.5 + _COST_CLASS * cc * 0.5
                + _COST_GIOU * cg + 100.0 * jnp.where(both, 0.0, 1.0))
        cost = cost + 10000.0 * jnp.where(fg, 0.0, 1.0)
        cost_s[sl, :] = cost
        cwork[sl, :] = cost
        iwork[sl, :] = ious
        iv, ii = fold_max(ious, gi, iv, ii)
        cv, ci = fold_min(cost, gi, cv, ci)
        return iv, ii, cv, ci

    iv, ii, cv, ci = jax.lax.fori_loop(
        0, NT, build, (neg_inf_v, q_idx, pos_inf_v, q_idx))

    s = iv
    cost_idxs = [ci]
    for _ in range(_OTA_K - 1):
        def extract(t, carry, ii=ii, ci=ci):
            iv2, ii2, cv2, ci2 = carry
            sl = pl.ds(t * _TQ, _TQ)
            gi = iota_tq + t * _TQ
            itile = jnp.where(gi == ii, -jnp.inf, iwork[sl, :])
            iwork[sl, :] = itile
            ctile = jnp.where(gi == ci, jnp.inf, cwork[sl, :])
            cwork[sl, :] = ctile
            iv2, ii2 = fold_max(itile, gi, iv2, ii2)
            cv2, ci2 = fold_min(ctile, gi, cv2, ci2)
            return iv2, ii2, cv2, ci2

        iv, ii, cv, ci = jax.lax.fori_loop(
            0, NT, extract, (neg_inf_v, q_idx, pos_inf_v, q_idx))
        s = s + iv
        cost_idxs.append(ci)
    dynamic_ks = jnp.maximum(s.astype(jnp.int32), 1)

    def row_resolve_terms(sl):
        cost = cost_s[sl, :]
        rmin = jnp.min(cost, axis=1, keepdims=True)
        row_argmin = jnp.min(jnp.where(cost == rmin, iota_g, G),
                             axis=1, keepdims=True)
        oh_row = (iota_g == row_argmin).astype(jnp.float32)
        return cost, oh_row

    def pass_ab(t, carry):
        colsum, pv, pi = carry
        sl = pl.ds(t * _TQ, _TQ)
        gi = iota_tq + t * _TQ
        m_pre = jnp.zeros((_TQ, G), jnp.float32)
        for j in range(_OTA_K):
            m_pre = jnp.where((gi == cost_idxs[j]) & (j < dynamic_ks),
                              1.0, m_pre)
        cost, oh_row = row_resolve_terms(sl)
        amg = jnp.sum(m_pre, axis=1, keepdims=True)
        newm = jnp.where(amg > 1, oh_row, m_pre)
        matching_ref[0, sl, :] = newm
        colsum = colsum + jnp.sum(newm, axis=0, keepdims=True)
        pen = cost + (amg > 0).astype(jnp.float32) * 100000.0
        pv, pi = fold_min(pen, gi, pv, pi)
        return colsum, pv, pi

    colsum, _, col_argmin = jax.lax.fori_loop(
        0, NT, pass_ab,
        (jnp.zeros((1, G), jnp.float32), pos_inf_v, q_idx))
    unmatched = colsum == 0

    def pass_c(t, carry):
        cmv, cmi = carry
        sl = pl.ds(t * _TQ, _TQ)
        gi = iota_tq + t * _TQ
        m_tile = matching_ref[0, sl, :]
        fix = (gi == col_argmin).astype(jnp.float32)
        m2 = jnp.where(unmatched, fix, m_tile)
        amg2 = jnp.sum(m2, axis=1, keepdims=True)
        cost, oh_row = row_resolve_terms(sl)
        m3 = jnp.where(amg2 > 1, oh_row, m2)
        matching_ref[0, sl, :] = m3
        sel = (jnp.sum(m3, axis=1, keepdims=True) > 0).astype(jnp.int32)
        rowmax = jnp.max(m3, axis=1, keepdims=True)
        mgt = jnp.min(jnp.where(m3 == rowmax, iota_g, G), axis=1, keepdims=True)
        selgt_ref[0, sl, 0:1] = sel
        selgt_ref[0, sl, 1:2] = mgt
        cmask = jnp.where(m3 > 0, cost, 1e18)
        tv = jnp.min(cmask, axis=0, keepdims=True)
        ti = jnp.min(jnp.where(cmask == tv, gi, Q), axis=0, keepdims=True)
        better = tv < cmv
        nv = jnp.where(better, tv, cmv)
        ni = jnp.where(better, ti, jnp.where(tv == cmv, jnp.minimum(cmi, ti), cmi))
        return nv, ni

    _, mqidx = jax.lax.fori_loop(
        0, NT, pass_c,
        (jnp.full((1, G), jnp.inf, jnp.float32), jnp.full((1, G), Q, jnp.int32)))
    mqidx_ref[0] = mqidx


def kernel(pred_logits, pred_boxes, pred_scores, tgt_labels, tgt_boxes_xyxy,
           image_size_xyxy):
    B, Q, C = pred_logits.shape
    B2 = B // 2
    G = tgt_labels.shape[1]

    lp = jnp.swapaxes(pred_logits[:B2], 1, 2)
    lc = jnp.swapaxes(pred_logits[B2:], 1, 2)
    pq = jnp.concatenate(
        [pred_boxes[:B2], pred_boxes[B2:], pred_scores], axis=-1)
    idp = tgt_labels[:B2].reshape(B2, 1, G)
    idc = tgt_labels[B2:].reshape(B2, 1, G)
    gp = jnp.swapaxes(tgt_boxes_xyxy[:B2], 1, 2)
    gc = jnp.swapaxes(tgt_boxes_xyxy[B2:], 1, 2)
    imp = image_size_xyxy[:B2].reshape(B2, 1, 4)
    imc = image_size_xyxy[B2:].reshape(B2, 1, 4)

    def spec(shape):
        n = len(shape)
        return pl.BlockSpec((1,) + shape[1:], lambda b: (b,) + (0,) * (n - 1))

    out_shapes = (
        jax.ShapeDtypeStruct((B2, Q, G), jnp.float32),
        jax.ShapeDtypeStruct((B2, Q, 2), jnp.int32),
        jax.ShapeDtypeStruct((B2, 1, G), jnp.int32),
    )
    args = (lp, lc, pq, idp, idc, gp, gc, imp, imc)
    matching, selgt, mqidx = pl.pallas_call(
        _body,
        grid=(B2,),
        in_specs=[spec(a.shape) for a in args],
        out_specs=tuple(spec(s.shape) for s in out_shapes),
        out_shape=out_shapes,
        scratch_shapes=[pltpu.VMEM((Q, G), jnp.float32),
                        pltpu.VMEM((Q, G), jnp.float32),
                        pltpu.VMEM((Q, G), jnp.float32)],
        compiler_params=pltpu.CompilerParams(
            dimension_semantics=("parallel",)),
    )(*args)

    return (matching,
            selgt[:, :, 0].astype(bool),
            selgt[:, :, 1],
            mqidx.reshape(B2, G))


# speedup vs baseline: 1.1668x; 1.0047x over previous
import jax
import jax.numpy as jnp
from jax.experimental import pallas as pl
from jax.experimental.pallas import tpu as pltpu

_COST_CLASS = 2.0
_COST_BBOX = 5.0
_COST_GIOU = 2.0
_ALPHA = 0.25
_GAMMA = 2.0
_OTA_K = 5
_TQ = 1024


def _gt_terms(gtT, img):
    gx0, gy0, gx1, gy1 = gtT[0:1, :], gtT[1:2, :], gtT[2:3, :], gtT[3:4, :]
    gcx = (gx0 + gx1) * 0.5
    gcy = (gy0 + gy1) * 0.5
    gw = gx1 - gx0
    gh = gy1 - gy0
    xx0 = gcx - 0.5 * gw
    yy0 = gcy - 0.5 * gh
    xx1 = gcx + 0.5 * gw
    yy1 = gcy + 0.5 * gh
    area_b = (gx1 - gx0) * (gy1 - gy0)
    i0 = img[0:1, 0:1]
    i1 = img[0:1, 1:2]
    i2 = img[0:1, 2:3]
    i3 = img[0:1, 3:4]
    tb = (gx0 / i0, gy0 / i1, gx1 / i2, gy1 / i3)
    return (gx0, gy0, gx1, gy1, gcx, gcy, xx0, yy0, xx1, yy1, area_b, tb,
            i0, i1, i2, i3)


def _frame_tile(bx, score, lgT, oh, g):
    (gx0, gy0, gx1, gy1, gcx, gcy, xx0, yy0, xx1, yy1, area_b, tb,
     i0, i1, i2, i3) = g
    x0, y0, x1, y1 = bx[:, 0:1], bx[:, 1:2], bx[:, 2:3], bx[:, 3:4]
    cx = (x0 + x1) * 0.5
    cy = (y0 + y1) * 0.5

    in_boxes = (cx > xx0) & (cx < xx1) & (cy > yy0) & (cy < yy1)
    in_boxes_all = jnp.sum(in_boxes.astype(jnp.int32), axis=1, keepdims=True) > 0
    r = 2.5
    w_rt = xx1 - xx0
    h_rt = yy1 - yy0
    in_centers = ((cx > gcx - r * w_rt) & (cx < gcx + r * w_rt)
                  & (cy > gcy - r * h_rt) & (cy < gcy + r * h_rt))
    in_centers_all = jnp.sum(in_centers.astype(jnp.int32), axis=1, keepdims=True) > 0
    fg = in_boxes_all | in_centers_all
    both = in_boxes & in_centers

    area_a = (x1 - x0) * (y1 - y0)
    lx = jnp.maximum(x0, gx0)
    ly = jnp.maximum(y0, gy0)
    rx = jnp.minimum(x1, gx1)
    ry = jnp.minimum(y1, gy1)
    iw = jnp.maximum(rx - lx, 0.0)
    ih = jnp.maximum(ry - ly, 0.0)
    inter = iw * ih
    union = area_a + area_b - inter
    iou = inter / (union + 1e-8)
    l2x = jnp.minimum(x0, gx0)
    l2y = jnp.minimum(y0, gy0)
    r2x = jnp.maximum(x1, gx1)
    r2y = jnp.maximum(y1, gy1)
    w2 = jnp.maximum(r2x - l2x, 0.0)
    h2 = jnp.maximum(r2y - l2y, 0.0)
    area_c = w2 * h2
    giou = iou - (area_c - union) / (area_c + 1e-8)

    cb = (jnp.abs(x0 / i0 - tb[0]) + jnp.abs(y0 / i1 - tb[1])
          + jnp.abs(x1 / i2 - tb[2]) + jnp.abs(y1 / i3 - tb[3]))

    lg = jax.lax.dot_general(
        lgT, oh, (((0,), (0,)), ((), ())),
        precision=jax.lax.Precision.HIGHEST,
        preferred_element_type=jnp.float32)
    p = jnp.sqrt(jax.nn.sigmoid(lg) * score)
    neg = (1.0 - _ALPHA) * (p * p) * -jnp.log(1.0 - p + 1e-8)
    pos = _ALPHA * ((1.0 - p) * (1.0 - p)) * -jnp.log(p + 1e-8)
    d = pos - neg
    return fg, both, iou, giou, cb, d


def _body(lp_ref, lc_ref, pq_ref, idp_ref, idc_ref,
          gp_ref, gc_ref, imp_ref, imc_ref,
          matching_ref, selgt_ref, mqidx_ref, cost_s):
    Q, C = lp_ref.shape[2], lp_ref.shape[1]
    G = idp_ref.shape[2]
    NT = Q // _TQ

    gterms_p = _gt_terms(gp_ref[0], imp_ref[0])
    gterms_c = _gt_terms(gc_ref[0], imc_ref[0])
    oh_p = (jax.lax.broadcasted_iota(jnp.int32, (C, G), 0)
            == idp_ref[0]).astype(jnp.float32)
    oh_c = (jax.lax.broadcasted_iota(jnp.int32, (C, G), 0)
            == idc_ref[0]).astype(jnp.float32)
    iota_tq = jax.lax.broadcasted_iota(jnp.int32, (_TQ, 1), 0)
    iota_g = jax.lax.broadcasted_iota(jnp.int32, (1, G), 1)

    def fold_min(tile, gi, v, i):
        tv = jnp.min(tile, axis=0, keepdims=True)
        ti = jnp.min(jnp.where(tile == tv, gi, Q), axis=0, keepdims=True)
        better = tv < v
        nv = jnp.where(better, tv, v)
        ni = jnp.where(better, ti, jnp.where(tv == v, jnp.minimum(i, ti), i))
        return nv, ni

    def tile_top5(tile, gi, is_min):
        cur = tile
        vals, idxs = [], []
        for _ in range(_OTA_K):
            if is_min:
                v = jnp.min(cur, axis=0, keepdims=True)
            else:
                v = jnp.max(cur, axis=0, keepdims=True)
            i = jnp.min(jnp.where(cur == v, gi, Q), axis=0, keepdims=True)
            cur = jnp.where(gi == i, jnp.inf if is_min else -jnp.inf, cur)
            vals.append(v)
            idxs.append(i)
        return jnp.concatenate(vals, 0), jnp.concatenate(idxs, 0)

    def merge_top5(va, ia, vb, ib, is_min):
        cv = jnp.concatenate([va, vb], 0)
        ci = jnp.concatenate([ia, ib], 0)
        vals, idxs = [], []
        for _ in range(_OTA_K):
            if is_min:
                v = jnp.min(cv, axis=0, keepdims=True)
            else:
                v = jnp.max(cv, axis=0, keepdims=True)
            i = jnp.min(jnp.where(cv == v, ci, Q), axis=0, keepdims=True)
            cv = jnp.where((cv == v) & (ci == i),
                           jnp.inf if is_min else -jnp.inf, cv)
            vals.append(v)
            idxs.append(i)
        return jnp.concatenate(vals, 0), jnp.concatenate(idxs, 0)

    pos_inf_v = jnp.full((1, G), jnp.inf, jnp.float32)
    q_idx = jnp.full((1, G), Q, jnp.int32)
    init_i5 = jnp.full((_OTA_K, G), Q, jnp.int32)

    def build(t, carry):
        iv5, ii5, cv5, ci5 = carry
        sl = pl.ds(t * _TQ, _TQ)
        gi = iota_tq + t * _TQ
        bx_p = pq_ref[0, sl, 0:4]
        bx_c = pq_ref[0, sl, 4:8]
        score = pq_ref[0, sl, 8:9]
        fg_p, both_p, iou_p, giou_p, cb_p, d_p = _frame_tile(
            bx_p, score, lp_ref[0, :, sl], oh_p, gterms_p)
        fg_c, both_c, iou_c, giou_c, cb_c, d_c = _frame_tile(
            bx_c, score, lc_ref[0, :, sl], oh_c, gterms_c)
        fg = fg_p & fg_c
        both = both_p & both_c
        ious = (iou_p + iou_c) * 0.5
        cc = d_p + d_c
        cg = -0.5 * (giou_p + giou_c)
        cost = (_COST_BBOX * (cb_p + cb_c) * 0.5 + _COST_CLASS * cc * 0.5
                + _COST_GIOU * cg + 100.0 * jnp.where(both, 0.0, 1.0))
        cost = cost + 10000.0 * jnp.where(fg, 0.0, 1.0)
        cost_s[sl, :] = cost
        tv, ti = tile_top5(ious, gi, is_min=False)
        iv5, ii5 = merge_top5(iv5, ii5, tv, ti, is_min=False)
        tv, ti = tile_top5(cost, gi, is_min=True)
        cv5, ci5 = merge_top5(cv5, ci5, tv, ti, is_min=True)
        return iv5, ii5, cv5, ci5

    iv5, ii5, cv5, ci5 = jax.lax.fori_loop(
        0, NT, build,
        (jnp.full((_OTA_K, G), -jnp.inf, jnp.float32), init_i5,
         jnp.full((_OTA_K, G), jnp.inf, jnp.float32), init_i5))

    s = iv5[0:1, :]
    for j in range(1, _OTA_K):
        s = s + iv5[j:j + 1, :]
    dynamic_ks = jnp.maximum(s.astype(jnp.int32), 1)
    cost_idxs = [ci5[j:j + 1, :] for j in range(_OTA_K)]

    def row_resolve_terms(sl):
        cost = cost_s[sl, :]
        rmin = jnp.min(cost, axis=1, keepdims=True)
        row_argmin = jnp.min(jnp.where(cost == rmin, iota_g, G),
                             axis=1, keepdims=True)
        oh_row = (iota_g == row_argmin).astype(jnp.float32)
        return cost, oh_row

    def pass_ab(t, carry):
        colsum, pv, pi = carry
        sl = pl.ds(t * _TQ, _TQ)
        gi = iota_tq + t * _TQ
        m_pre = jnp.zeros((_TQ, G), jnp.float32)
        for j in range(_OTA_K):
            m_pre = jnp.where((gi == cost_idxs[j]) & (j < dynamic_ks),
                              1.0, m_pre)
        cost, oh_row = row_resolve_terms(sl)
        amg = jnp.sum(m_pre, axis=1, keepdims=True)
        newm = jnp.where(amg > 1, oh_row, m_pre)
        matching_ref[0, sl, :] = newm
        colsum = colsum + jnp.sum(newm, axis=0, keepdims=True)
        pen = cost + (amg > 0).astype(jnp.float32) * 100000.0
        pv, pi = fold_min(pen, gi, pv, pi)
        return colsum, pv, pi

    colsum, _, col_argmin = jax.lax.fori_loop(
        0, NT, pass_ab,
        (jnp.zeros((1, G), jnp.float32), pos_inf_v, q_idx))
    unmatched = colsum == 0

    def pass_c(t, carry):
        cmv, cmi = carry
        sl = pl.ds(t * _TQ, _TQ)
        gi = iota_tq + t * _TQ
        m_tile = matching_ref[0, sl, :]
        fix = (gi == col_argmin).astype(jnp.float32)
        m2 = jnp.where(unmatched, fix, m_tile)
        amg2 = jnp.sum(m2, axis=1, keepdims=True)
        cost, oh_row = row_resolve_terms(sl)
        m3 = jnp.where(amg2 > 1, oh_row, m2)
        matching_ref[0, sl, :] = m3
        sel = (jnp.sum(m3, axis=1, keepdims=True) > 0).astype(jnp.int32)
        rowmax = jnp.max(m3, axis=1, keepdims=True)
        mgt = jnp.min(jnp.where(m3 == rowmax, iota_g, G), axis=1, keepdims=True)
        selgt_ref[0, sl, 0:1] = sel
        selgt_ref[0, sl, 1:2] = mgt
        cmask = jnp.where(m3 > 0, cost, 1e18)
        tv = jnp.min(cmask, axis=0, keepdims=True)
        ti = jnp.min(jnp.where(cmask == tv, gi, Q), axis=0, keepdims=True)
        better = tv < cmv
        nv = jnp.where(better, tv, cmv)
        ni = jnp.where(better, ti, jnp.where(tv == cmv, jnp.minimum(cmi, ti), cmi))
        return nv, ni

    _, mqidx = jax.lax.fori_loop(
        0, NT, pass_c,
        (jnp.full((1, G), jnp.inf, jnp.float32), jnp.full((1, G), Q, jnp.int32)))
    mqidx_ref[0] = mqidx


def kernel(pred_logits, pred_boxes, pred_scores, tgt_labels, tgt_boxes_xyxy,
           image_size_xyxy):
    B, Q, C = pred_logits.shape
    B2 = B // 2
    G = tgt_labels.shape[1]

    lp = jnp.swapaxes(pred_logits[:B2], 1, 2)
    lc = jnp.swapaxes(pred_logits[B2:], 1, 2)
    pq = jnp.concatenate(
        [pred_boxes[:B2], pred_boxes[B2:], pred_scores], axis=-1)
    idp = tgt_labels[:B2].reshape(B2, 1, G)
    idc = tgt_labels[B2:].reshape(B2, 1, G)
    gp = jnp.swapaxes(tgt_boxes_xyxy[:B2], 1, 2)
    gc = jnp.swapaxes(tgt_boxes_xyxy[B2:], 1, 2)
    imp = image_size_xyxy[:B2].reshape(B2, 1, 4)
    imc = image_size_xyxy[B2:].reshape(B2, 1, 4)

    def spec(shape):
        n = len(shape)
        return pl.BlockSpec((1,) + shape[1:], lambda b: (b,) + (0,) * (n - 1))

    out_shapes = (
        jax.ShapeDtypeStruct((B2, Q, G), jnp.float32),
        jax.ShapeDtypeStruct((B2, Q, 2), jnp.int32),
        jax.ShapeDtypeStruct((B2, 1, G), jnp.int32),
    )
    args = (lp, lc, pq, idp, idc, gp, gc, imp, imc)
    matching, selgt, mqidx = pl.pallas_call(
        _body,
        grid=(B2,),
        in_specs=[spec(a.shape) for a in args],
        out_specs=tuple(spec(s.shape) for s in out_shapes),
        out_shape=out_shapes,
        scratch_shapes=[pltpu.VMEM((Q, G), jnp.float32)],
        compiler_params=pltpu.CompilerParams(
            dimension_semantics=("parallel",)),
    )(*args)

    return (matching,
            selgt[:, :, 0].astype(bool),
            selgt[:, :, 1],
            mqidx.reshape(B2, G))


# speedup vs baseline: 1.1692x; 1.0021x over previous
import jax
import jax.numpy as jnp
from jax.experimental import pallas as pl
from jax.experimental.pallas import tpu as pltpu

_COST_CLASS = 2.0
_COST_BBOX = 5.0
_COST_GIOU = 2.0
_ALPHA = 0.25
_GAMMA = 2.0
_OTA_K = 5
_TQ = 1024


def _gt_terms(gtT, img):
    gx0, gy0, gx1, gy1 = gtT[0:1, :], gtT[1:2, :], gtT[2:3, :], gtT[3:4, :]
    gcx = (gx0 + gx1) * 0.5
    gcy = (gy0 + gy1) * 0.5
    gw = gx1 - gx0
    gh = gy1 - gy0
    xx0 = gcx - 0.5 * gw
    yy0 = gcy - 0.5 * gh
    xx1 = gcx + 0.5 * gw
    yy1 = gcy + 0.5 * gh
    area_b = (gx1 - gx0) * (gy1 - gy0)
    i0 = img[0:1, 0:1]
    i1 = img[0:1, 1:2]
    i2 = img[0:1, 2:3]
    i3 = img[0:1, 3:4]
    tb = (gx0 / i0, gy0 / i1, gx1 / i2, gy1 / i3)
    return (gx0, gy0, gx1, gy1, gcx, gcy, xx0, yy0, xx1, yy1, area_b, tb,
            i0, i1, i2, i3)


def _frame_tile(bx, score, lgT, oh, g):
    (gx0, gy0, gx1, gy1, gcx, gcy, xx0, yy0, xx1, yy1, area_b, tb,
     i0, i1, i2, i3) = g
    x0, y0, x1, y1 = bx[:, 0:1], bx[:, 1:2], bx[:, 2:3], bx[:, 3:4]
    cx = (x0 + x1) * 0.5
    cy = (y0 + y1) * 0.5

    in_boxes = (cx > xx0) & (cx < xx1) & (cy > yy0) & (cy < yy1)
    in_boxes_all = jnp.sum(in_boxes.astype(jnp.int32), axis=1, keepdims=True) > 0
    r = 2.5
    w_rt = xx1 - xx0
    h_rt = yy1 - yy0
    in_centers = ((cx > gcx - r * w_rt) & (cx < gcx + r * w_rt)
                  & (cy > gcy - r * h_rt) & (cy < gcy + r * h_rt))
    in_centers_all = jnp.sum(in_centers.astype(jnp.int32), axis=1, keepdims=True) > 0
    fg = in_boxes_all | in_centers_all
    both = in_boxes & in_centers

    area_a = (x1 - x0) * (y1 - y0)
    lx = jnp.maximum(x0, gx0)
    ly = jnp.maximum(y0, gy0)
    rx = jnp.minimum(x1, gx1)
    ry = jnp.minimum(y1, gy1)
    iw = jnp.maximum(rx - lx, 0.0)
    ih = jnp.maximum(ry - ly, 0.0)
    inter = iw * ih
    union = area_a + area_b - inter
    iou = inter / (union + 1e-8)
    l2x = jnp.minimum(x0, gx0)
    l2y = jnp.minimum(y0, gy0)
    r2x = jnp.maximum(x1, gx1)
    r2y = jnp.maximum(y1, gy1)
    w2 = jnp.maximum(r2x - l2x, 0.0)
    h2 = jnp.maximum(r2y - l2y, 0.0)
    area_c = w2 * h2
    giou = iou - (area_c - union) / (area_c + 1e-8)

    cb = (jnp.abs(x0 / i0 - tb[0]) + jnp.abs(y0 / i1 - tb[1])
          + jnp.abs(x1 / i2 - tb[2]) + jnp.abs(y1 / i3 - tb[3]))

    hi = lgT.astype(jnp.bfloat16)
    t1 = lgT - hi.astype(jnp.float32)
    mid = t1.astype(jnp.bfloat16)
    lo = (t1 - mid.astype(jnp.float32)).astype(jnp.bfloat16)
    dn = (((0,), (0,)), ((), ()))

    def bmm(a, b):
        return jax.lax.dot_general(a, b, dn,
                                   preferred_element_type=jnp.float32)

    lg = bmm(hi, oh) + bmm(mid, oh) + bmm(lo, oh)
    p = jnp.sqrt(jax.nn.sigmoid(lg) * score)
    neg = (1.0 - _ALPHA) * (p * p) * -jnp.log(1.0 - p + 1e-8)
    pos = _ALPHA * ((1.0 - p) * (1.0 - p)) * -jnp.log(p + 1e-8)
    d = pos - neg
    return fg, both, iou, giou, cb, d


def _body(lp_ref, lc_ref, pq_ref, idp_ref, idc_ref,
          gp_ref, gc_ref, imp_ref, imc_ref,
          matching_ref, selgt_ref, mqidx_ref, cost_s):
    Q, C = lp_ref.shape[2], lp_ref.shape[1]
    G = idp_ref.shape[2]
    NT = Q // _TQ

    gterms_p = _gt_terms(gp_ref[0], imp_ref[0])
    gterms_c = _gt_terms(gc_ref[0], imc_ref[0])
    oh_p = (jax.lax.broadcasted_iota(jnp.int32, (C, G), 0)
            == idp_ref[0]).astype(jnp.bfloat16)
    oh_c = (jax.lax.broadcasted_iota(jnp.int32, (C, G), 0)
            == idc_ref[0]).astype(jnp.bfloat16)
    iota_tq = jax.lax.broadcasted_iota(jnp.int32, (_TQ, 1), 0)
    iota_g = jax.lax.broadcasted_iota(jnp.int32, (1, G), 1)

    def fold_min(tile, gi, v, i):
        tv = jnp.min(tile, axis=0, keepdims=True)
        ti = jnp.min(jnp.where(tile == tv, gi, Q), axis=0, keepdims=True)
        better = tv < v
        nv = jnp.where(better, tv, v)
        ni = jnp.where(better, ti, jnp.where(tv == v, jnp.minimum(i, ti), i))
        return nv, ni

    def tile_top5(tile, gi, is_min):
        cur = tile
        vals, idxs = [], []
        for _ in range(_OTA_K):
            if is_min:
                v = jnp.min(cur, axis=0, keepdims=True)
            else:
                v = jnp.max(cur, axis=0, keepdims=True)
            i = jnp.min(jnp.where(cur == v, gi, Q), axis=0, keepdims=True)
            cur = jnp.where(gi == i, jnp.inf if is_min else -jnp.inf, cur)
            vals.append(v)
            idxs.append(i)
        return jnp.concatenate(vals, 0), jnp.concatenate(idxs, 0)

    def merge_top5(va, ia, vb, ib, is_min):
        cv = jnp.concatenate([va, vb], 0)
        ci = jnp.concatenate([ia, ib], 0)
        vals, idxs = [], []
        for _ in range(_OTA_K):
            if is_min:
                v = jnp.min(cv, axis=0, keepdims=True)
            else:
                v = jnp.max(cv, axis=0, keepdims=True)
            i = jnp.min(jnp.where(cv == v, ci, Q), axis=0, keepdims=True)
            cv = jnp.where((cv == v) & (ci == i),
                           jnp.inf if is_min else -jnp.inf, cv)
            vals.append(v)
            idxs.append(i)
        return jnp.concatenate(vals, 0), jnp.concatenate(idxs, 0)

    pos_inf_v = jnp.full((1, G), jnp.inf, jnp.float32)
    q_idx = jnp.full((1, G), Q, jnp.int32)
    init_i5 = jnp.full((_OTA_K, G), Q, jnp.int32)

    def build(t, carry):
        iv5, ii5, cv5, ci5 = carry
        sl = pl.ds(t * _TQ, _TQ)
        gi = iota_tq + t * _TQ
        bx_p = pq_ref[0, sl, 0:4]
        bx_c = pq_ref[0, sl, 4:8]
        score = pq_ref[0, sl, 8:9]
        fg_p, both_p, iou_p, giou_p, cb_p, d_p = _frame_tile(
            bx_p, score, lp_ref[0, :, sl], oh_p, gterms_p)
        fg_c, both_c, iou_c, giou_c, cb_c, d_c = _frame_tile(
            bx_c, score, lc_ref[0, :, sl], oh_c, gterms_c)
        fg = fg_p & fg_c
        both = both_p & both_c
        ious = (iou_p + iou_c) * 0.5
        cc = d_p + d_c
        cg = -0.5 * (giou_p + giou_c)
        cost = (_COST_BBOX * (cb_p + cb_c) * 0.5 + _COST_CLASS * cc * 0.5
                + _COST_GIOU * cg + 100.0 * jnp.where(both, 0.0, 1.0))
        cost = cost + 10000.0 * jnp.where(fg, 0.0, 1.0)
        cost_s[sl, :] = cost
        tv, ti = tile_top5(ious, gi, is_min=False)
        iv5, ii5 = merge_top5(iv5, ii5, tv, ti, is_min=False)
        tv, ti = tile_top5(cost, gi, is_min=True)
        cv5, ci5 = merge_top5(cv5, ci5, tv, ti, is_min=True)
        return iv5, ii5, cv5, ci5

    iv5, ii5, cv5, ci5 = jax.lax.fori_loop(
        0, NT, build,
        (jnp.full((_OTA_K, G), -jnp.inf, jnp.float32), init_i5,
         jnp.full((_OTA_K, G), jnp.inf, jnp.float32), init_i5))

    s = iv5[0:1, :]
    for j in range(1, _OTA_K):
        s = s + iv5[j:j + 1, :]
    dynamic_ks = jnp.maximum(s.astype(jnp.int32), 1)
    cost_idxs = [ci5[j:j + 1, :] for j in range(_OTA_K)]

    def row_resolve_terms(sl):
        cost = cost_s[sl, :]
        rmin = jnp.min(cost, axis=1, keepdims=True)
        row_argmin = jnp.min(jnp.where(cost == rmin, iota_g, G),
                             axis=1, keepdims=True)
        oh_row = (iota_g == row_argmin).astype(jnp.float32)
        return cost, oh_row

    def pass_ab(t, carry):
        colsum, pv, pi = carry
        sl = pl.ds(t * _TQ, _TQ)
        gi = iota_tq + t * _TQ
        m_pre = jnp.zeros((_TQ, G), jnp.float32)
        for j in range(_OTA_K):
            m_pre = jnp.where((gi == cost_idxs[j]) & (j < dynamic_ks),
                              1.0, m_pre)
        cost, oh_row = row_resolve_terms(sl)
        amg = jnp.sum(m_pre, axis=1, keepdims=True)
        newm = jnp.where(amg > 1, oh_row, m_pre)
        matching_ref[0, sl, :] = newm
        colsum = colsum + jnp.sum(newm, axis=0, keepdims=True)
        pen = cost + (amg > 0).astype(jnp.float32) * 100000.0
        pv, pi = fold_min(pen, gi, pv, pi)
        return colsum, pv, pi

    colsum, _, col_argmin = jax.lax.fori_loop(
        0, NT, pass_ab,
        (jnp.zeros((1, G), jnp.float32), pos_inf_v, q_idx))
    unmatched = colsum == 0

    def pass_c(t, carry):
        cmv, cmi = carry
        sl = pl.ds(t * _TQ, _TQ)
        gi = iota_tq + t * _TQ
        m_tile = matching_ref[0, sl, :]
        fix = (gi == col_argmin).astype(jnp.float32)
        m2 = jnp.where(unmatched, fix, m_tile)
        amg2 = jnp.sum(m2, axis=1, keepdims=True)
        cost, oh_row = row_resolve_terms(sl)
        m3 = jnp.where(amg2 > 1, oh_row, m2)
        matching_ref[0, sl, :] = m3
        sel = (jnp.sum(m3, axis=1, keepdims=True) > 0).astype(jnp.int32)
        rowmax = jnp.max(m3, axis=1, keepdims=True)
        mgt = jnp.min(jnp.where(m3 == rowmax, iota_g, G), axis=1, keepdims=True)
        selgt_ref[0, sl, 0:1] = sel
        selgt_ref[0, sl, 1:2] = mgt
        cmask = jnp.where(m3 > 0, cost, 1e18)
        tv = jnp.min(cmask, axis=0, keepdims=True)
        ti = jnp.min(jnp.where(cmask == tv, gi, Q), axis=0, keepdims=True)
        better = tv < cmv
        nv = jnp.where(better, tv, cmv)
        ni = jnp.where(better, ti, jnp.where(tv == cmv, jnp.minimum(cmi, ti), cmi))
        return nv, ni

    _, mqidx = jax.lax.fori_loop(
        0, NT, pass_c,
        (jnp.full((1, G), jnp.inf, jnp.float32), jnp.full((1, G), Q, jnp.int32)))
    mqidx_ref[0] = mqidx


def kernel(pred_logits, pred_boxes, pred_scores, tgt_labels, tgt_boxes_xyxy,
           image_size_xyxy):
    B, Q, C = pred_logits.shape
    B2 = B // 2
    G = tgt_labels.shape[1]

    lp = jnp.swapaxes(pred_logits[:B2], 1, 2)
    lc = jnp.swapaxes(pred_logits[B2:], 1, 2)
    pq = jnp.concatenate(
        [pred_boxes[:B2], pred_boxes[B2:], pred_scores], axis=-1)
    idp = tgt_labels[:B2].reshape(B2, 1, G)
    idc = tgt_labels[B2:].reshape(B2, 1, G)
    gp = jnp.swapaxes(tgt_boxes_xyxy[:B2], 1, 2)
    gc = jnp.swapaxes(tgt_boxes_xyxy[B2:], 1, 2)
    imp = image_size_xyxy[:B2].reshape(B2, 1, 4)
    imc = image_size_xyxy[B2:].reshape(B2, 1, 4)

    def spec(shape):
        n = len(shape)
        return pl.BlockSpec((1,) + shape[1:], lambda b: (b,) + (0,) * (n - 1))

    out_shapes = (
        jax.ShapeDtypeStruct((B2, Q, G), jnp.float32),
        jax.ShapeDtypeStruct((B2, Q, 2), jnp.int32),
        jax.ShapeDtypeStruct((B2, 1, G), jnp.int32),
    )
    args = (lp, lc, pq, idp, idc, gp, gc, imp, imc)
    matching, selgt, mqidx = pl.pallas_call(
        _body,
        grid=(B2,),
        in_specs=[spec(a.shape) for a in args],
        out_specs=tuple(spec(s.shape) for s in out_shapes),
        out_shape=out_shapes,
        scratch_shapes=[pltpu.VMEM((Q, G), jnp.float32)],
        compiler_params=pltpu.CompilerParams(
            dimension_semantics=("parallel",)),
    )(*args)

    return (matching,
            selgt[:, :, 0].astype(bool),
            selgt[:, :, 1],
            mqidx.reshape(B2, G))


# speedup vs baseline: 1.1696x; 1.0003x over previous
import jax
import jax.numpy as jnp
from jax.experimental import pallas as pl
from jax.experimental.pallas import tpu as pltpu

_COST_CLASS = 2.0
_COST_BBOX = 5.0
_COST_GIOU = 2.0
_ALPHA = 0.25
_GAMMA = 2.0
_OTA_K = 5
_TQ = 1024


def _gt_terms(gtT, img):
    gx0, gy0, gx1, gy1 = gtT[0:1, :], gtT[1:2, :], gtT[2:3, :], gtT[3:4, :]
    gcx = (gx0 + gx1) * 0.5
    gcy = (gy0 + gy1) * 0.5
    gw = gx1 - gx0
    gh = gy1 - gy0
    xx0 = gcx - 0.5 * gw
    yy0 = gcy - 0.5 * gh
    xx1 = gcx + 0.5 * gw
    yy1 = gcy + 0.5 * gh
    area_b = (gx1 - gx0) * (gy1 - gy0)
    i0 = img[0:1, 0:1]
    i1 = img[0:1, 1:2]
    i2 = img[0:1, 2:3]
    i3 = img[0:1, 3:4]
    tb = (gx0 / i0, gy0 / i1, gx1 / i2, gy1 / i3)
    return (gx0, gy0, gx1, gy1, gcx, gcy, xx0, yy0, xx1, yy1, area_b, tb,
            i0, i1, i2, i3)


def _frame_tile(bx, score, lgT, oh, g):
    (gx0, gy0, gx1, gy1, gcx, gcy, xx0, yy0, xx1, yy1, area_b, tb,
     i0, i1, i2, i3) = g
    x0, y0, x1, y1 = bx[:, 0:1], bx[:, 1:2], bx[:, 2:3], bx[:, 3:4]
    cx = (x0 + x1) * 0.5
    cy = (y0 + y1) * 0.5

    in_boxes = (cx > xx0) & (cx < xx1) & (cy > yy0) & (cy < yy1)
    in_boxes_all = jnp.sum(in_boxes.astype(jnp.int32), axis=1, keepdims=True) > 0
    r = 2.5
    w_rt = xx1 - xx0
    h_rt = yy1 - yy0
    in_centers = ((cx > gcx - r * w_rt) & (cx < gcx + r * w_rt)
                  & (cy > gcy - r * h_rt) & (cy < gcy + r * h_rt))
    in_centers_all = jnp.sum(in_centers.astype(jnp.int32), axis=1, keepdims=True) > 0
    fg = in_boxes_all | in_centers_all
    both = in_boxes & in_centers

    area_a = (x1 - x0) * (y1 - y0)
    lx = jnp.maximum(x0, gx0)
    ly = jnp.maximum(y0, gy0)
    rx = jnp.minimum(x1, gx1)
    ry = jnp.minimum(y1, gy1)
    iw = jnp.maximum(rx - lx, 0.0)
    ih = jnp.maximum(ry - ly, 0.0)
    inter = iw * ih
    union = area_a + area_b - inter
    iou = inter / (union + 1e-8)
    l2x = jnp.minimum(x0, gx0)
    l2y = jnp.minimum(y0, gy0)
    r2x = jnp.maximum(x1, gx1)
    r2y = jnp.maximum(y1, gy1)
    w2 = jnp.maximum(r2x - l2x, 0.0)
    h2 = jnp.maximum(r2y - l2y, 0.0)
    area_c = w2 * h2
    giou = iou - (area_c - union) / (area_c + 1e-8)

    cb = (jnp.abs(x0 / i0 - tb[0]) + jnp.abs(y0 / i1 - tb[1])
          + jnp.abs(x1 / i2 - tb[2]) + jnp.abs(y1 / i3 - tb[3]))

    hi = lgT.astype(jnp.bfloat16)
    t1 = lgT - hi.astype(jnp.float32)
    mid = t1.astype(jnp.bfloat16)
    lo = (t1 - mid.astype(jnp.float32)).astype(jnp.bfloat16)
    dn = (((0,), (0,)), ((), ()))

    def bmm(a, b):
        return jax.lax.dot_general(a, b, dn,
                                   preferred_element_type=jnp.float32)

    lg = bmm(hi, oh) + bmm(mid, oh) + bmm(lo, oh)
    p = jnp.sqrt(jax.nn.sigmoid(lg) * score)
    neg = (1.0 - _ALPHA) * (p * p) * -jnp.log(1.0 - p + 1e-8)
    pos = _ALPHA * ((1.0 - p) * (1.0 - p)) * -jnp.log(p + 1e-8)
    d = pos - neg
    return fg, both, iou, giou, cb, d


def _body(lp_ref, lc_ref, pq_ref, idp_ref, idc_ref,
          gp_ref, gc_ref, imp_ref, imc_ref,
          matching_ref, selgt_ref, mqidx_ref, cost_s):
    Q, C = lp_ref.shape[2], lp_ref.shape[1]
    G = idp_ref.shape[2]
    NT = Q // _TQ

    gterms_p = _gt_terms(gp_ref[0], imp_ref[0])
    gterms_c = _gt_terms(gc_ref[0], imc_ref[0])
    oh_p = (jax.lax.broadcasted_iota(jnp.int32, (C, G), 0)
            == idp_ref[0]).astype(jnp.bfloat16)
    oh_c = (jax.lax.broadcasted_iota(jnp.int32, (C, G), 0)
            == idc_ref[0]).astype(jnp.bfloat16)
    iota_tq = jax.lax.broadcasted_iota(jnp.int32, (_TQ, 1), 0)
    iota_g = jax.lax.broadcasted_iota(jnp.int32, (1, G), 1)

    def fold_min(tile, gi, v, i):
        tv = jnp.min(tile, axis=0, keepdims=True)
        ti = jnp.min(jnp.where(tile == tv, gi, Q), axis=0, keepdims=True)
        better = tv < v
        nv = jnp.where(better, tv, v)
        ni = jnp.where(better, ti, jnp.where(tv == v, jnp.minimum(i, ti), i))
        return nv, ni

    def tile_top5(tile, gi, is_min):
        cur = tile
        vals, idxs = [], []
        for j in range(_OTA_K):
            if is_min:
                v = jnp.min(cur, axis=0, keepdims=True)
            else:
                v = jnp.max(cur, axis=0, keepdims=True)
            i = jnp.min(jnp.where(cur == v, gi, Q), axis=0, keepdims=True)
            if j < _OTA_K - 1:
                cur = jnp.where(gi == i, jnp.inf if is_min else -jnp.inf, cur)
            vals.append(v)
            idxs.append(i)
        return jnp.concatenate(vals, 0), jnp.concatenate(idxs, 0)

    def merge_top5(va, ia, vb, ib, is_min):
        cv = jnp.concatenate([va, vb], 0)
        ci = jnp.concatenate([ia, ib], 0)
        vals, idxs = [], []
        for j in range(_OTA_K):
            if is_min:
                v = jnp.min(cv, axis=0, keepdims=True)
            else:
                v = jnp.max(cv, axis=0, keepdims=True)
            i = jnp.min(jnp.where(cv == v, ci, Q), axis=0, keepdims=True)
            if j < _OTA_K - 1:
                cv = jnp.where((cv == v) & (ci == i),
                               jnp.inf if is_min else -jnp.inf, cv)
            vals.append(v)
            idxs.append(i)
        return jnp.concatenate(vals, 0), jnp.concatenate(idxs, 0)

    pos_inf_v = jnp.full((1, G), jnp.inf, jnp.float32)
    q_idx = jnp.full((1, G), Q, jnp.int32)
    init_i5 = jnp.full((_OTA_K, G), Q, jnp.int32)

    def build(t, carry):
        iv5, ii5, cv5, ci5 = carry
        sl = pl.ds(t * _TQ, _TQ)
        gi = iota_tq + t * _TQ
        bx_p = pq_ref[0, sl, 0:4]
        bx_c = pq_ref[0, sl, 4:8]
        score = pq_ref[0, sl, 8:9]
        fg_p, both_p, iou_p, giou_p, cb_p, d_p = _frame_tile(
            bx_p, score, lp_ref[0, :, sl], oh_p, gterms_p)
        fg_c, both_c, iou_c, giou_c, cb_c, d_c = _frame_tile(
            bx_c, score, lc_ref[0, :, sl], oh_c, gterms_c)
        fg = fg_p & fg_c
        both = both_p & both_c
        ious = (iou_p + iou_c) * 0.5
        cc = d_p + d_c
        cg = -0.5 * (giou_p + giou_c)
        cost = (_COST_BBOX * (cb_p + cb_c) * 0.5 + _COST_CLASS * cc * 0.5
                + _COST_GIOU * cg + 100.0 * jnp.where(both, 0.0, 1.0))
        cost = cost + 10000.0 * jnp.where(fg, 0.0, 1.0)
        cost_s[sl, :] = cost
        tv, ti = tile_top5(ious, gi, is_min=False)
        iv5, ii5 = merge_top5(iv5, ii5, tv, ti, is_min=False)
        tv, ti = tile_top5(cost, gi, is_min=True)
        cv5, ci5 = merge_top5(cv5, ci5, tv, ti, is_min=True)
        return iv5, ii5, cv5, ci5

    iv5, ii5, cv5, ci5 = jax.lax.fori_loop(
        0, NT, build,
        (jnp.full((_OTA_K, G), -jnp.inf, jnp.float32), init_i5,
         jnp.full((_OTA_K, G), jnp.inf, jnp.float32), init_i5))

    s = iv5[0:1, :]
    for j in range(1, _OTA_K):
        s = s + iv5[j:j + 1, :]
    dynamic_ks = jnp.maximum(s.astype(jnp.int32), 1)
    cost_idxs = [ci5[j:j + 1, :] for j in range(_OTA_K)]

    def row_resolve_terms(sl):
        cost = cost_s[sl, :]
        rmin = jnp.min(cost, axis=1, keepdims=True)
        row_argmin = jnp.min(jnp.where(cost == rmin, iota_g, G),
                             axis=1, keepdims=True)
        oh_row = (iota_g == row_argmin).astype(jnp.float32)
        return cost, oh_row

    def pass_ab(t, carry):
        colsum, pv, pi = carry
        sl = pl.ds(t * _TQ, _TQ)
        gi = iota_tq + t * _TQ
        m_pre = jnp.zeros((_TQ, G), jnp.float32)
        for j in range(_OTA_K):
            m_pre = jnp.where((gi == cost_idxs[j]) & (j < dynamic_ks),
                              1.0, m_pre)
        cost, oh_row = row_resolve_terms(sl)
        amg = jnp.sum(m_pre, axis=1, keepdims=True)
        newm = jnp.where(amg > 1, oh_row, m_pre)
        matching_ref[0, sl, :] = newm
        colsum = colsum + jnp.sum(newm, axis=0, keepdims=True)
        pen = cost + (amg > 0).astype(jnp.float32) * 100000.0
        pv, pi = fold_min(pen, gi, pv, pi)
        return colsum, pv, pi

    colsum, _, col_argmin = jax.lax.fori_loop(
        0, NT, pass_ab,
        (jnp.zeros((1, G), jnp.float32), pos_inf_v, q_idx))
    unmatched = colsum == 0

    def pass_c(t, carry):
        cmv, cmi = carry
        sl = pl.ds(t * _TQ, _TQ)
        gi = iota_tq + t * _TQ
        m_tile = matching_ref[0, sl, :]
        fix = (gi == col_argmin).astype(jnp.float32)
        m2 = jnp.where(unmatched, fix, m_tile)
        amg2 = jnp.sum(m2, axis=1, keepdims=True)
        cost, oh_row = row_resolve_terms(sl)
        m3 = jnp.where(amg2 > 1, oh_row, m2)
        matching_ref[0, sl, :] = m3
        sel = (jnp.sum(m3, axis=1, keepdims=True) > 0).astype(jnp.int32)
        rowmax = jnp.max(m3, axis=1, keepdims=True)
        mgt = jnp.min(jnp.where(m3 == rowmax, iota_g, G), axis=1, keepdims=True)
        selgt_ref[0, sl, 0:1] = sel
        selgt_ref[0, sl, 1:2] = mgt
        cmask = jnp.where(m3 > 0, cost, 1e18)
        tv = jnp.min(cmask, axis=0, keepdims=True)
        ti = jnp.min(jnp.where(cmask == tv, gi, Q), axis=0, keepdims=True)
        better = tv < cmv
        nv = jnp.where(better, tv, cmv)
        ni = jnp.where(better, ti, jnp.where(tv == cmv, jnp.minimum(cmi, ti), cmi))
        return nv, ni

    _, mqidx = jax.lax.fori_loop(
        0, NT, pass_c,
        (jnp.full((1, G), jnp.inf, jnp.float32), jnp.full((1, G), Q, jnp.int32)))
    mqidx_ref[0] = mqidx


def kernel(pred_logits, pred_boxes, pred_scores, tgt_labels, tgt_boxes_xyxy,
           image_size_xyxy):
    B, Q, C = pred_logits.shape
    B2 = B // 2
    G = tgt_labels.shape[1]

    lp = jnp.swapaxes(pred_logits[:B2], 1, 2)
    lc = jnp.swapaxes(pred_logits[B2:], 1, 2)
    pq = jnp.concatenate(
        [pred_boxes[:B2], pred_boxes[B2:], pred_scores], axis=-1)
    idp = tgt_labels[:B2].reshape(B2, 1, G)
    idc = tgt_labels[B2:].reshape(B2, 1, G)
    gp = jnp.swapaxes(tgt_boxes_xyxy[:B2], 1, 2)
    gc = jnp.swapaxes(tgt_boxes_xyxy[B2:], 1, 2)
    imp = image_size_xyxy[:B2].reshape(B2, 1, 4)
    imc = image_size_xyxy[B2:].reshape(B2, 1, 4)

    def spec(shape):
        n = len(shape)
        return pl.BlockSpec((1,) + shape[1:], lambda b: (b,) + (0,) * (n - 1))

    out_shapes = (
        jax.ShapeDtypeStruct((B2, Q, G), jnp.float32),
        jax.ShapeDtypeStruct((B2, Q, 2), jnp.int32),
        jax.ShapeDtypeStruct((B2, 1, G), jnp.int32),
    )
    args = (lp, lc, pq, idp, idc, gp, gc, imp, imc)
    matching, selgt, mqidx = pl.pallas_call(
        _body,
        grid=(B2,),
        in_specs=[spec(a.shape) for a in args],
        out_specs=tuple(spec(s.shape) for s in out_shapes),
        out_shape=out_shapes,
        scratch_shapes=[pltpu.VMEM((Q, G), jnp.float32)],
        compiler_params=pltpu.CompilerParams(
            dimension_semantics=("parallel",)),
    )(*args)

    return (matching,
            selgt[:, :, 0].astype(bool),
            selgt[:, :, 1],
            mqidx.reshape(B2, G))


# speedup vs baseline: 1.4034x; 1.1999x over previous
import jax
import jax.numpy as jnp
from jax.experimental import pallas as pl
from jax.experimental.pallas import tpu as pltpu

_COST_CLASS = 2.0
_COST_BBOX = 5.0
_COST_GIOU = 2.0
_ALPHA = 0.25
_GAMMA = 2.0
_OTA_K = 5
_TQ = 1024


def _gt_terms(gtT, img):
    gx0, gy0, gx1, gy1 = gtT[0:1, :], gtT[1:2, :], gtT[2:3, :], gtT[3:4, :]
    gcx = (gx0 + gx1) * 0.5
    gcy = (gy0 + gy1) * 0.5
    gw = gx1 - gx0
    gh = gy1 - gy0
    xx0 = gcx - 0.5 * gw
    yy0 = gcy - 0.5 * gh
    xx1 = gcx + 0.5 * gw
    yy1 = gcy + 0.5 * gh
    area_b = (gx1 - gx0) * (gy1 - gy0)
    i0 = img[0:1, 0:1]
    i1 = img[0:1, 1:2]
    i2 = img[0:1, 2:3]
    i3 = img[0:1, 3:4]
    tb = (gx0 / i0, gy0 / i1, gx1 / i2, gy1 / i3)
    return (gx0, gy0, gx1, gy1, gcx, gcy, xx0, yy0, xx1, yy1, area_b, tb,
            i0, i1, i2, i3)


def _pair_tile(bx_p, bx_c, score, lgT2, oh2, gpk, lane_lo, G):
    (gx0, gy0, gx1, gy1, gcx, gcy, xx0, yy0, xx1, yy1, area_b, tb,
     i0, i1, i2, i3) = gpk

    def packq(a, b):
        return jnp.where(lane_lo, a, b)

    x0 = packq(bx_p[:, 0:1], bx_c[:, 0:1])
    y0 = packq(bx_p[:, 1:2], bx_c[:, 1:2])
    x1 = packq(bx_p[:, 2:3], bx_c[:, 2:3])
    y1 = packq(bx_p[:, 3:4], bx_c[:, 3:4])
    cx = (x0 + x1) * 0.5
    cy = (y0 + y1) * 0.5

    in_boxes = (cx > xx0) & (cx < xx1) & (cy > yy0) & (cy < yy1)
    r = 2.5
    w_rt = xx1 - xx0
    h_rt = yy1 - yy0
    in_centers = ((cx > gcx - r * w_rt) & (cx < gcx + r * w_rt)
                  & (cy > gcy - r * h_rt) & (cy < gcy + r * h_rt))
    ib_p, ib_c = in_boxes[:, 0:G], in_boxes[:, G:2 * G]
    ic_p, ic_c = in_centers[:, 0:G], in_centers[:, G:2 * G]
    fg_p = ((jnp.sum(ib_p.astype(jnp.int32), axis=1, keepdims=True) > 0)
            | (jnp.sum(ic_p.astype(jnp.int32), axis=1, keepdims=True) > 0))
    fg_c = ((jnp.sum(ib_c.astype(jnp.int32), axis=1, keepdims=True) > 0)
            | (jnp.sum(ic_c.astype(jnp.int32), axis=1, keepdims=True) > 0))
    fg = fg_p & fg_c
    bic = in_boxes & in_centers
    both = bic[:, 0:G] & bic[:, G:2 * G]

    area_a = (x1 - x0) * (y1 - y0)
    lx = jnp.maximum(x0, gx0)
    ly = jnp.maximum(y0, gy0)
    rx = jnp.minimum(x1, gx1)
    ry = jnp.minimum(y1, gy1)
    iw = jnp.maximum(rx - lx, 0.0)
    ih = jnp.maximum(ry - ly, 0.0)
    inter = iw * ih
    union = area_a + area_b - inter
    iou = inter / (union + 1e-8)
    l2x = jnp.minimum(x0, gx0)
    l2y = jnp.minimum(y0, gy0)
    r2x = jnp.maximum(x1, gx1)
    r2y = jnp.maximum(y1, gy1)
    w2 = jnp.maximum(r2x - l2x, 0.0)
    h2 = jnp.maximum(r2y - l2y, 0.0)
    area_c = w2 * h2
    giou = iou - (area_c - union) / (area_c + 1e-8)

    cb = (jnp.abs(x0 / i0 - tb[0]) + jnp.abs(y0 / i1 - tb[1])
          + jnp.abs(x1 / i2 - tb[2]) + jnp.abs(y1 / i3 - tb[3]))

    hi = lgT2.astype(jnp.bfloat16)
    t1 = lgT2 - hi.astype(jnp.float32)
    mid = t1.astype(jnp.bfloat16)
    lo = (t1 - mid.astype(jnp.float32)).astype(jnp.bfloat16)
    dn = (((0,), (0,)), ((), ()))

    def bmm(a, b):
        return jax.lax.dot_general(a, b, dn,
                                   preferred_element_type=jnp.float32)

    lg = bmm(hi, oh2) + bmm(mid, oh2) + bmm(lo, oh2)
    p = jnp.sqrt(jax.nn.sigmoid(lg) * score)
    neg = (1.0 - _ALPHA) * (p * p) * -jnp.log(1.0 - p + 1e-8)
    pos = _ALPHA * ((1.0 - p) * (1.0 - p)) * -jnp.log(p + 1e-8)
    d = pos - neg

    ious = (iou[:, 0:G] + iou[:, G:2 * G]) * 0.5
    cc = d[:, 0:G] + d[:, G:2 * G]
    cg = -0.5 * (giou[:, 0:G] + giou[:, G:2 * G])
    cbs = cb[:, 0:G] + cb[:, G:2 * G]
    return fg, both, ious, cc, cg, cbs


def _body(lp_ref, lc_ref, pq_ref, idp_ref, idc_ref,
          gp_ref, gc_ref, imp_ref, imc_ref,
          matching_ref, selgt_ref, mqidx_ref, cost_s):
    Q, C = lp_ref.shape[2], lp_ref.shape[1]
    G = idp_ref.shape[2]
    NT = Q // _TQ

    gterms_p = _gt_terms(gp_ref[0], imp_ref[0])
    gterms_c = _gt_terms(gc_ref[0], imc_ref[0])

    def pk(a, b):
        return jnp.concatenate([a, b], axis=1)

    ones_g = jnp.ones((1, G), jnp.float32)
    gpk = []
    for k in range(16):
        a, b = gterms_p[k], gterms_c[k]
        if k == 11:
            gpk.append(tuple(pk(a[m], b[m]) for m in range(4)))
        elif k >= 12:
            gpk.append(pk(a * ones_g, b * ones_g))
        else:
            gpk.append(pk(a, b))
    gpk = tuple(gpk)

    ids2 = pk(idp_ref[0], idc_ref[0] + C)
    oh2 = (jax.lax.broadcasted_iota(jnp.int32, (2 * C, 2 * G), 0)
           == ids2).astype(jnp.bfloat16)
    lane_lo = jax.lax.broadcasted_iota(jnp.int32, (1, 2 * G), 1) < G
    iota_tq = jax.lax.broadcasted_iota(jnp.int32, (_TQ, 1), 0)
    iota_g = jax.lax.broadcasted_iota(jnp.int32, (1, G), 1)

    def fold_min(tile, gi, v, i):
        tv = jnp.min(tile, axis=0, keepdims=True)
        ti = jnp.min(jnp.where(tile == tv, gi, Q), axis=0, keepdims=True)
        better = tv < v
        nv = jnp.where(better, tv, v)
        ni = jnp.where(better, ti, jnp.where(tv == v, jnp.minimum(i, ti), i))
        return nv, ni

    def tile_top5(tile, gi, is_min):
        cur = tile
        vals, idxs = [], []
        for j in range(_OTA_K):
            if is_min:
                v = jnp.min(cur, axis=0, keepdims=True)
            else:
                v = jnp.max(cur, axis=0, keepdims=True)
            i = jnp.min(jnp.where(cur == v, gi, Q), axis=0, keepdims=True)
            if j < _OTA_K - 1:
                cur = jnp.where(gi == i, jnp.inf if is_min else -jnp.inf, cur)
            vals.append(v)
            idxs.append(i)
        return jnp.concatenate(vals, 0), jnp.concatenate(idxs, 0)

    def merge_top5(va, ia, vb, ib, is_min):
        cv = jnp.concatenate([va, vb], 0)
        ci = jnp.concatenate([ia, ib], 0)
        vals, idxs = [], []
        for j in range(_OTA_K):
            if is_min:
                v = jnp.min(cv, axis=0, keepdims=True)
            else:
                v = jnp.max(cv, axis=0, keepdims=True)
            i = jnp.min(jnp.where(cv == v, ci, Q), axis=0, keepdims=True)
            if j < _OTA_K - 1:
                cv = jnp.where((cv == v) & (ci == i),
                               jnp.inf if is_min else -jnp.inf, cv)
            vals.append(v)
            idxs.append(i)
        return jnp.concatenate(vals, 0), jnp.concatenate(idxs, 0)

    pos_inf_v = jnp.full((1, G), jnp.inf, jnp.float32)
    q_idx = jnp.full((1, G), Q, jnp.int32)
    init_i5 = jnp.full((_OTA_K, G), Q, jnp.int32)

    def build(t, carry):
        iv5, ii5, cv5, ci5 = carry
        sl = pl.ds(t * _TQ, _TQ)
        gi = iota_tq + t * _TQ
        bx_p = pq_ref[0, sl, 0:4]
        bx_c = pq_ref[0, sl, 4:8]
        score = pq_ref[0, sl, 8:9]
        lgT2 = jnp.concatenate([lp_ref[0, :, sl], lc_ref[0, :, sl]], axis=0)
        fg, both, ious, cc, cg, cbs = _pair_tile(
            bx_p, bx_c, score, lgT2, oh2, gpk, lane_lo, G)
        cost = (_COST_BBOX * cbs * 0.5 + _COST_CLASS * cc * 0.5
                + _COST_GIOU * cg + 100.0 * jnp.where(both, 0.0, 1.0))
        cost = cost + 10000.0 * jnp.where(fg, 0.0, 1.0)
        cost_s[sl, :] = cost
        tv, ti = tile_top5(ious, gi, is_min=False)
        iv5, ii5 = merge_top5(iv5, ii5, tv, ti, is_min=False)
        tv, ti = tile_top5(cost, gi, is_min=True)
        cv5, ci5 = merge_top5(cv5, ci5, tv, ti, is_min=True)
        return iv5, ii5, cv5, ci5

    iv5, ii5, cv5, ci5 = jax.lax.fori_loop(
        0, NT, build,
        (jnp.full((_OTA_K, G), -jnp.inf, jnp.float32), init_i5,
         jnp.full((_OTA_K, G), jnp.inf, jnp.float32), init_i5))

    s = iv5[0:1, :]
    for j in range(1, _OTA_K):
        s = s + iv5[j:j + 1, :]
    dynamic_ks = jnp.maximum(s.astype(jnp.int32), 1)
    cost_idxs = [ci5[j:j + 1, :] for j in range(_OTA_K)]

    def row_resolve_terms(sl):
        cost = cost_s[sl, :]
        rmin = jnp.min(cost, axis=1, keepdims=True)
        row_argmin = jnp.min(jnp.where(cost == rmin, iota_g, G),
                             axis=1, keepdims=True)
        oh_row = (iota_g == row_argmin).astype(jnp.float32)
        return cost, oh_row

    def pass_ab(t, carry):
        colsum, pv, pi = carry
        sl = pl.ds(t * _TQ, _TQ)
        gi = iota_tq + t * _TQ
        m_pre = jnp.zeros((_TQ, G), jnp.float32)
        for j in range(_OTA_K):
            m_pre = jnp.where((gi == cost_idxs[j]) & (j < dynamic_ks),
                              1.0, m_pre)
        cost, oh_row = row_resolve_terms(sl)
        amg = jnp.sum(m_pre, axis=1, keepdims=True)
        newm = jnp.where(amg > 1, oh_row, m_pre)
        matching_ref[0, sl, :] = newm
        colsum = colsum + jnp.sum(newm, axis=0, keepdims=True)
        pen = cost + (amg > 0).astype(jnp.float32) * 100000.0
        pv, pi = fold_min(pen, gi, pv, pi)
        return colsum, pv, pi

    colsum, _, col_argmin = jax.lax.fori_loop(
        0, NT, pass_ab,
        (jnp.zeros((1, G), jnp.float32), pos_inf_v, q_idx))
    unmatched = colsum == 0

    def pass_c(t, carry):
        cmv, cmi = carry
        sl = pl.ds(t * _TQ, _TQ)
        gi = iota_tq + t * _TQ
        m_tile = matching_ref[0, sl, :]
        fix = (gi == col_argmin).astype(jnp.float32)
        m2 = jnp.where(unmatched, fix, m_tile)
        amg2 = jnp.sum(m2, axis=1, keepdims=True)
        cost, oh_row = row_resolve_terms(sl)
        m3 = jnp.where(amg2 > 1, oh_row, m2)
        matching_ref[0, sl, :] = m3
        sel = (jnp.sum(m3, axis=1, keepdims=True) > 0).astype(jnp.int32)
        rowmax = jnp.max(m3, axis=1, keepdims=True)
        mgt = jnp.min(jnp.where(m3 == rowmax, iota_g, G), axis=1, keepdims=True)
        selgt_ref[0, sl, 0:1] = sel
        selgt_ref[0, sl, 1:2] = mgt
        cmask = jnp.where(m3 > 0, cost, 1e18)
        tv = jnp.min(cmask, axis=0, keepdims=True)
        ti = jnp.min(jnp.where(cmask == tv, gi, Q), axis=0, keepdims=True)
        better = tv < cmv
        nv = jnp.where(better, tv, cmv)
        ni = jnp.where(better, ti, jnp.where(tv == cmv, jnp.minimum(cmi, ti), cmi))
        return nv, ni

    _, mqidx = jax.lax.fori_loop(
        0, NT, pass_c,
        (jnp.full((1, G), jnp.inf, jnp.float32), jnp.full((1, G), Q, jnp.int32)))
    mqidx_ref[0] = mqidx


def kernel(pred_logits, pred_boxes, pred_scores, tgt_labels, tgt_boxes_xyxy,
           image_size_xyxy):
    B, Q, C = pred_logits.shape
    B2 = B // 2
    G = tgt_labels.shape[1]

    lp = jnp.swapaxes(pred_logits[:B2], 1, 2)
    lc = jnp.swapaxes(pred_logits[B2:], 1, 2)
    pq = jnp.concatenate(
        [pred_boxes[:B2], pred_boxes[B2:], pred_scores], axis=-1)
    idp = tgt_labels[:B2].reshape(B2, 1, G)
    idc = tgt_labels[B2:].reshape(B2, 1, G)
    gp = jnp.swapaxes(tgt_boxes_xyxy[:B2], 1, 2)
    gc = jnp.swapaxes(tgt_boxes_xyxy[B2:], 1, 2)
    imp = image_size_xyxy[:B2].reshape(B2, 1, 4)
    imc = image_size_xyxy[B2:].reshape(B2, 1, 4)

    def spec(shape):
        n = len(shape)
        return pl.BlockSpec((1,) + shape[1:], lambda b: (b,) + (0,) * (n - 1))

    out_shapes = (
        jax.ShapeDtypeStruct((B2, Q, G), jnp.float32),
        jax.ShapeDtypeStruct((B2, Q, 2), jnp.int32),
        jax.ShapeDtypeStruct((B2, 1, G), jnp.int32),
    )
    args = (lp, lc, pq, idp, idc, gp, gc, imp, imc)
    matching, selgt, mqidx = pl.pallas_call(
        _body,
        grid=(B2,),
        in_specs=[spec(a.shape) for a in args],
        out_specs=tuple(spec(s.shape) for s in out_shapes),
        out_shape=out_shapes,
        scratch_shapes=[pltpu.VMEM((Q, G), jnp.float32)],
        compiler_params=pltpu.CompilerParams(
            dimension_semantics=("parallel",)),
    )(*args)

    return (matching,
            selgt[:, :, 0].astype(bool),
            selgt[:, :, 1],
            mqidx.reshape(B2, G))


# speedup vs baseline: 1.4536x; 1.0358x over previous
import jax
import jax.numpy as jnp
from jax.experimental import pallas as pl
from jax.experimental.pallas import tpu as pltpu

_COST_CLASS = 2.0
_COST_BBOX = 5.0
_COST_GIOU = 2.0
_ALPHA = 0.25
_GAMMA = 2.0
_OTA_K = 5
_TQ = 2048


def _gt_terms(gtT, img):
    gx0, gy0, gx1, gy1 = gtT[0:1, :], gtT[1:2, :], gtT[2:3, :], gtT[3:4, :]
    gcx = (gx0 + gx1) * 0.5
    gcy = (gy0 + gy1) * 0.5
    gw = gx1 - gx0
    gh = gy1 - gy0
    xx0 = gcx - 0.5 * gw
    yy0 = gcy - 0.5 * gh
    xx1 = gcx + 0.5 * gw
    yy1 = gcy + 0.5 * gh
    area_b = (gx1 - gx0) * (gy1 - gy0)
    i0 = img[0:1, 0:1]
    i1 = img[0:1, 1:2]
    i2 = img[0:1, 2:3]
    i3 = img[0:1, 3:4]
    tb = (gx0 / i0, gy0 / i1, gx1 / i2, gy1 / i3)
    return (gx0, gy0, gx1, gy1, gcx, gcy, xx0, yy0, xx1, yy1, area_b, tb,
            i0, i1, i2, i3)


def _pair_tile(bx_p, bx_c, score, lgT2, oh2, gpk, lane_lo, G):
    (gx0, gy0, gx1, gy1, gcx, gcy, xx0, yy0, xx1, yy1, area_b, tb,
     i0, i1, i2, i3) = gpk

    def packq(a, b):
        return jnp.where(lane_lo, a, b)

    x0 = packq(bx_p[:, 0:1], bx_c[:, 0:1])
    y0 = packq(bx_p[:, 1:2], bx_c[:, 1:2])
    x1 = packq(bx_p[:, 2:3], bx_c[:, 2:3])
    y1 = packq(bx_p[:, 3:4], bx_c[:, 3:4])
    cx = (x0 + x1) * 0.5
    cy = (y0 + y1) * 0.5

    in_boxes = (cx > xx0) & (cx < xx1) & (cy > yy0) & (cy < yy1)
    r = 2.5
    w_rt = xx1 - xx0
    h_rt = yy1 - yy0
    in_centers = ((cx > gcx - r * w_rt) & (cx < gcx + r * w_rt)
                  & (cy > gcy - r * h_rt) & (cy < gcy + r * h_rt))
    ib_p, ib_c = in_boxes[:, 0:G], in_boxes[:, G:2 * G]
    ic_p, ic_c = in_centers[:, 0:G], in_centers[:, G:2 * G]
    fg_p = ((jnp.sum(ib_p.astype(jnp.int32), axis=1, keepdims=True) > 0)
            | (jnp.sum(ic_p.astype(jnp.int32), axis=1, keepdims=True) > 0))
    fg_c = ((jnp.sum(ib_c.astype(jnp.int32), axis=1, keepdims=True) > 0)
            | (jnp.sum(ic_c.astype(jnp.int32), axis=1, keepdims=True) > 0))
    fg = fg_p & fg_c
    bic = in_boxes & in_centers
    both = bic[:, 0:G] & bic[:, G:2 * G]

    area_a = (x1 - x0) * (y1 - y0)
    lx = jnp.maximum(x0, gx0)
    ly = jnp.maximum(y0, gy0)
    rx = jnp.minimum(x1, gx1)
    ry = jnp.minimum(y1, gy1)
    iw = jnp.maximum(rx - lx, 0.0)
    ih = jnp.maximum(ry - ly, 0.0)
    inter = iw * ih
    union = area_a + area_b - inter
    iou = inter / (union + 1e-8)
    l2x = jnp.minimum(x0, gx0)
    l2y = jnp.minimum(y0, gy0)
    r2x = jnp.maximum(x1, gx1)
    r2y = jnp.maximum(y1, gy1)
    w2 = jnp.maximum(r2x - l2x, 0.0)
    h2 = jnp.maximum(r2y - l2y, 0.0)
    area_c = w2 * h2
    giou = iou - (area_c - union) / (area_c + 1e-8)

    cb = (jnp.abs(x0 / i0 - tb[0]) + jnp.abs(y0 / i1 - tb[1])
          + jnp.abs(x1 / i2 - tb[2]) + jnp.abs(y1 / i3 - tb[3]))

    hi = lgT2.astype(jnp.bfloat16)
    t1 = lgT2 - hi.astype(jnp.float32)
    mid = t1.astype(jnp.bfloat16)
    lo = (t1 - mid.astype(jnp.float32)).astype(jnp.bfloat16)
    dn = (((0,), (0,)), ((), ()))

    def bmm(a, b):
        return jax.lax.dot_general(a, b, dn,
                                   preferred_element_type=jnp.float32)

    lg = bmm(hi, oh2) + bmm(mid, oh2) + bmm(lo, oh2)
    p = jnp.sqrt(jax.nn.sigmoid(lg) * score)
    neg = (1.0 - _ALPHA) * (p * p) * -jnp.log(1.0 - p + 1e-8)
    pos = _ALPHA * ((1.0 - p) * (1.0 - p)) * -jnp.log(p + 1e-8)
    d = pos - neg

    ious = (iou[:, 0:G] + iou[:, G:2 * G]) * 0.5
    cc = d[:, 0:G] + d[:, G:2 * G]
    cg = -0.5 * (giou[:, 0:G] + giou[:, G:2 * G])
    cbs = cb[:, 0:G] + cb[:, G:2 * G]
    return fg, both, ious, cc, cg, cbs


def _body(lp_ref, lc_ref, pq_ref, idp_ref, idc_ref,
          gp_ref, gc_ref, imp_ref, imc_ref,
          matching_ref, selgt_ref, mqidx_ref, cost_s):
    Q, C = lp_ref.shape[2], lp_ref.shape[1]
    G = idp_ref.shape[2]
    NT = Q // _TQ

    gterms_p = _gt_terms(gp_ref[0], imp_ref[0])
    gterms_c = _gt_terms(gc_ref[0], imc_ref[0])

    def pk(a, b):
        return jnp.concatenate([a, b], axis=1)

    ones_g = jnp.ones((1, G), jnp.float32)
    gpk = []
    for k in range(16):
        a, b = gterms_p[k], gterms_c[k]
        if k == 11:
            gpk.append(tuple(pk(a[m], b[m]) for m in range(4)))
        elif k >= 12:
            gpk.append(pk(a * ones_g, b * ones_g))
        else:
            gpk.append(pk(a, b))
    gpk = tuple(gpk)

    ids2 = pk(idp_ref[0], idc_ref[0] + C)
    oh2 = (jax.lax.broadcasted_iota(jnp.int32, (2 * C, 2 * G), 0)
           == ids2).astype(jnp.bfloat16)
    lane_lo = jax.lax.broadcasted_iota(jnp.int32, (1, 2 * G), 1) < G
    iota_tq = jax.lax.broadcasted_iota(jnp.int32, (_TQ, 1), 0)
    iota_g = jax.lax.broadcasted_iota(jnp.int32, (1, G), 1)

    def fold_min(tile, gi, v, i):
        tv = jnp.min(tile, axis=0, keepdims=True)
        ti = jnp.min(jnp.where(tile == tv, gi, Q), axis=0, keepdims=True)
        better = tv < v
        nv = jnp.where(better, tv, v)
        ni = jnp.where(better, ti, jnp.where(tv == v, jnp.minimum(i, ti), i))
        return nv, ni

    def tile_top5(tile, gi, is_min):
        cur = tile
        vals, idxs = [], []
        for j in range(_OTA_K):
            if is_min:
                v = jnp.min(cur, axis=0, keepdims=True)
            else:
                v = jnp.max(cur, axis=0, keepdims=True)
            i = jnp.min(jnp.where(cur == v, gi, Q), axis=0, keepdims=True)
            if j < _OTA_K - 1:
                cur = jnp.where(gi == i, jnp.inf if is_min else -jnp.inf, cur)
            vals.append(v)
            idxs.append(i)
        return jnp.concatenate(vals, 0), jnp.concatenate(idxs, 0)

    def merge_top5(va, ia, vb, ib, is_min):
        cv = jnp.concatenate([va, vb], 0)
        ci = jnp.concatenate([ia, ib], 0)
        vals, idxs = [], []
        for j in range(_OTA_K):
            if is_min:
                v = jnp.min(cv, axis=0, keepdims=True)
            else:
                v = jnp.max(cv, axis=0, keepdims=True)
            i = jnp.min(jnp.where(cv == v, ci, Q), axis=0, keepdims=True)
            if j < _OTA_K - 1:
                cv = jnp.where((cv == v) & (ci == i),
                               jnp.inf if is_min else -jnp.inf, cv)
            vals.append(v)
            idxs.append(i)
        return jnp.concatenate(vals, 0), jnp.concatenate(idxs, 0)

    pos_inf_v = jnp.full((1, G), jnp.inf, jnp.float32)
    q_idx = jnp.full((1, G), Q, jnp.int32)
    init_i5 = jnp.full((_OTA_K, G), Q, jnp.int32)

    def build(t, carry):
        iv5, ii5, cv5, ci5 = carry
        sl = pl.ds(t * _TQ, _TQ)
        gi = iota_tq + t * _TQ
        bx_p = pq_ref[0, sl, 0:4]
        bx_c = pq_ref[0, sl, 4:8]
        score = pq_ref[0, sl, 8:9]
        lgT2 = jnp.concatenate([lp_ref[0, :, sl], lc_ref[0, :, sl]], axis=0)
        fg, both, ious, cc, cg, cbs = _pair_tile(
            bx_p, bx_c, score, lgT2, oh2, gpk, lane_lo, G)
        cost = (_COST_BBOX * cbs * 0.5 + _COST_CLASS * cc * 0.5
                + _COST_GIOU * cg + 100.0 * jnp.where(both, 0.0, 1.0))
        cost = cost + 10000.0 * jnp.where(fg, 0.0, 1.0)
        cost_s[sl, :] = cost
        tv, ti = tile_top5(ious, gi, is_min=False)
        iv5, ii5 = merge_top5(iv5, ii5, tv, ti, is_min=False)
        tv, ti = tile_top5(cost, gi, is_min=True)
        cv5, ci5 = merge_top5(cv5, ci5, tv, ti, is_min=True)
        return iv5, ii5, cv5, ci5

    iv5, ii5, cv5, ci5 = jax.lax.fori_loop(
        0, NT, build,
        (jnp.full((_OTA_K, G), -jnp.inf, jnp.float32), init_i5,
         jnp.full((_OTA_K, G), jnp.inf, jnp.float32), init_i5))

    s = iv5[0:1, :]
    for j in range(1, _OTA_K):
        s = s + iv5[j:j + 1, :]
    dynamic_ks = jnp.maximum(s.astype(jnp.int32), 1)
    cost_idxs = [ci5[j:j + 1, :] for j in range(_OTA_K)]

    def row_resolve_terms(sl):
        cost = cost_s[sl, :]
        rmin = jnp.min(cost, axis=1, keepdims=True)
        row_argmin = jnp.min(jnp.where(cost == rmin, iota_g, G),
                             axis=1, keepdims=True)
        oh_row = (iota_g == row_argmin).astype(jnp.float32)
        return cost, oh_row

    def pass_ab(t, carry):
        colsum, pv, pi = carry
        sl = pl.ds(t * _TQ, _TQ)
        gi = iota_tq + t * _TQ
        m_pre = jnp.zeros((_TQ, G), jnp.float32)
        for j in range(_OTA_K):
            m_pre = jnp.where((gi == cost_idxs[j]) & (j < dynamic_ks),
                              1.0, m_pre)
        cost, oh_row = row_resolve_terms(sl)
        amg = jnp.sum(m_pre, axis=1, keepdims=True)
        newm = jnp.where(amg > 1, oh_row, m_pre)
        matching_ref[0, sl, :] = newm
        colsum = colsum + jnp.sum(newm, axis=0, keepdims=True)
        pen = cost + (amg > 0).astype(jnp.float32) * 100000.0
        pv, pi = fold_min(pen, gi, pv, pi)
        return colsum, pv, pi

    colsum, _, col_argmin = jax.lax.fori_loop(
        0, NT, pass_ab,
        (jnp.zeros((1, G), jnp.float32), pos_inf_v, q_idx))
    unmatched = colsum == 0

    def pass_c(t, carry):
        cmv, cmi = carry
        sl = pl.ds(t * _TQ, _TQ)
        gi = iota_tq + t * _TQ
        m_tile = matching_ref[0, sl, :]
        fix = (gi == col_argmin).astype(jnp.float32)
        m2 = jnp.where(unmatched, fix, m_tile)
        amg2 = jnp.sum(m2, axis=1, keepdims=True)
        cost, oh_row = row_resolve_terms(sl)
        m3 = jnp.where(amg2 > 1, oh_row, m2)
        matching_ref[0, sl, :] = m3
        sel = (jnp.sum(m3, axis=1, keepdims=True) > 0).astype(jnp.int32)
        rowmax = jnp.max(m3, axis=1, keepdims=True)
        mgt = jnp.min(jnp.where(m3 == rowmax, iota_g, G), axis=1, keepdims=True)
        selgt_ref[0, sl, 0:1] = sel
        selgt_ref[0, sl, 1:2] = mgt
        cmask = jnp.where(m3 > 0, cost, 1e18)
        tv = jnp.min(cmask, axis=0, keepdims=True)
        ti = jnp.min(jnp.where(cmask == tv, gi, Q), axis=0, keepdims=True)
        better = tv < cmv
        nv = jnp.where(better, tv, cmv)
        ni = jnp.where(better, ti, jnp.where(tv == cmv, jnp.minimum(cmi, ti), cmi))
        return nv, ni

    _, mqidx = jax.lax.fori_loop(
        0, NT, pass_c,
        (jnp.full((1, G), jnp.inf, jnp.float32), jnp.full((1, G), Q, jnp.int32)))
    mqidx_ref[0] = mqidx


def kernel(pred_logits, pred_boxes, pred_scores, tgt_labels, tgt_boxes_xyxy,
           image_size_xyxy):
    B, Q, C = pred_logits.shape
    B2 = B // 2
    G = tgt_labels.shape[1]

    lp = jnp.swapaxes(pred_logits[:B2], 1, 2)
    lc = jnp.swapaxes(pred_logits[B2:], 1, 2)
    pq = jnp.concatenate(
        [pred_boxes[:B2], pred_boxes[B2:], pred_scores], axis=-1)
    idp = tgt_labels[:B2].reshape(B2, 1, G)
    idc = tgt_labels[B2:].reshape(B2, 1, G)
    gp = jnp.swapaxes(tgt_boxes_xyxy[:B2], 1, 2)
    gc = jnp.swapaxes(tgt_boxes_xyxy[B2:], 1, 2)
    imp = image_size_xyxy[:B2].reshape(B2, 1, 4)
    imc = image_size_xyxy[B2:].reshape(B2, 1, 4)

    def spec(shape):
        n = len(shape)
        return pl.BlockSpec((1,) + shape[1:], lambda b: (b,) + (0,) * (n - 1))

    out_shapes = (
        jax.ShapeDtypeStruct((B2, Q, G), jnp.float32),
        jax.ShapeDtypeStruct((B2, Q, 2), jnp.int32),
        jax.ShapeDtypeStruct((B2, 1, G), jnp.int32),
    )
    args = (lp, lc, pq, idp, idc, gp, gc, imp, imc)
    matching, selgt, mqidx = pl.pallas_call(
        _body,
        grid=(B2,),
        in_specs=[spec(a.shape) for a in args],
        out_specs=tuple(spec(s.shape) for s in out_shapes),
        out_shape=out_shapes,
        scratch_shapes=[pltpu.VMEM((Q, G), jnp.float32)],
        compiler_params=pltpu.CompilerParams(
            dimension_semantics=("parallel",)),
    )(*args)

    return (matching,
            selgt[:, :, 0].astype(bool),
            selgt[:, :, 1],
            mqidx.reshape(B2, G))
